# Initial kernel scaffold; baseline (speedup 1.0000x reference)
import jax
import jax.numpy as jnp
from jax.experimental import pallas as pl


def kernel(x, q_Y_sample, adj, t, num_steps, W_t1, b_t1, W_t2, b_t2, Wr, Wg0, as0, ad0, bg0, Wg1, as1, ad1, bg1, Wg2, as2, ad2, bg2, Wf1, bf1, Wf2, bf2):
    raise NotImplementedError("write your pallas kernel here")


# speedup vs baseline: 20.1322x; 20.1322x over previous
import functools
import math

import jax
import jax.numpy as jnp
from jax import lax
from jax.experimental import pallas as pl
from jax.experimental.pallas import tpu as pltpu
from jax.experimental.pallas import tpu_sc as plsc

N = 10000
E = 320000
NFEAT = 128
NLABEL = 5
NHID = 128
NHEAD = 6
DIN = NFEAT + NLABEL
DREL = DIN + 1
FDIM = NHID + NLABEL
HW = NHEAD * NHID
HP = 256
LP = 16

NC, NS, L = 2, 16, 16
NWORK = NC * NS
EPW = E // NWORK
K = 80
NCHUNK = EPW // K
K2 = 40
NCHUNK2 = EPW // K2
ALN = 624
TAIL = N - NS * ALN

BLK = 2000
GRID = N // BLK


def _prep_body(pe_abs, w1, b1, w2, b2, pe_rel, wr, t_abs_o, tflat_o):
    z = jnp.dot(pe_abs[...], w1[...], preferred_element_type=jnp.float32)
    z = z + b1[...]
    z = jnp.where(z > 0, z, jnp.exp(z) - 1.0)
    ta = jnp.dot(z, w2[...], preferred_element_type=jnp.float32) + b2[...]
    t_abs_o[...] = ta
    tflat_o[...] = jnp.dot(pe_rel[...], wr[...],
                           preferred_element_type=jnp.float32)


def _prep(pe_abs, w1, b1, w2, b2, pe_rel, wr):
    return pl.pallas_call(
        _prep_body,
        out_shape=(jax.ShapeDtypeStruct((8, NHID), jnp.float32),
                   jax.ShapeDtypeStruct((8, HW), jnp.float32)),
    )(pe_abs, w1, b1, w2, b2, pe_rel, wr)


def _matmul_tail(hblk, wp, asn, adn, tflat, mask, hw_o, s1_o, ad_o):
    hw = jnp.dot(hblk, wp[...], preferred_element_type=jnp.float32)
    hw_o[...] = hw
    svec = asn[0:1, :] + tflat[0:1, :]
    dvec = adn[0:1, :]
    s1_o[...] = jnp.dot(hw * svec, mask[...],
                        preferred_element_type=jnp.float32)
    ad_o[...] = jnp.dot(hw * dvec, mask[...],
                        preferred_element_type=jnp.float32)


def _layer0_body(hpad, wp, asn, adn, tflat, mask, hw_o, s1_o, ad_o):
    _matmul_tail(hpad[...], wp, asn, adn, tflat, mask, hw_o, s1_o, ad_o)


def _layern_body(u0, u1, bt, qpad, wp, asn, adn, tflat, mask,
                 hw_o, s1_o, ad_o):
    g = (u0[...] + u1[...]) * (1.0 / NHEAD) + bt[0:1, :]
    e = jnp.where(g > 0, g, jnp.exp(g) - 1.0)
    hblk = jnp.concatenate([e, qpad[...]], axis=1)
    _matmul_tail(hblk, wp, asn, adn, tflat, mask, hw_o, s1_o, ad_o)


def _row_spec(w):
    return pl.BlockSpec((BLK, w), lambda i: (i, 0))


def _full_spec(r, c):
    return pl.BlockSpec((r, c), lambda i: (0, 0))


_LAYER_OUT = (jax.ShapeDtypeStruct((N, HW), jnp.float32),
              jax.ShapeDtypeStruct((N, LP), jnp.float32),
              jax.ShapeDtypeStruct((N, LP), jnp.float32))
_LAYER_OUT_SPECS = (_row_spec(HW), _row_spec(LP), _row_spec(LP))


def _layer0(hpad, wp, asn, adn, tflat, mask):
    return pl.pallas_call(
        _layer0_body,
        grid=(GRID,),
        in_specs=[_row_spec(HP), _full_spec(HP, HW), _full_spec(8, HW),
                  _full_spec(8, HW), _full_spec(8, HW), _full_spec(HW, LP)],
        out_specs=_LAYER_OUT_SPECS,
        out_shape=_LAYER_OUT,
    )(hpad, wp, asn, adn, tflat, mask)


def _layern(u0, u1, bt, qpad, wp, asn, adn, tflat, mask):
    return pl.pallas_call(
        _layern_body,
        grid=(GRID,),
        in_specs=[_row_spec(NHID), _row_spec(NHID), _full_spec(8, NHID),
                  _row_spec(NHID), _full_spec(HP, HW), _full_spec(8, HW),
                  _full_spec(8, HW), _full_spec(8, HW), _full_spec(HW, LP)],
        out_specs=_LAYER_OUT_SPECS,
        out_shape=_LAYER_OUT,
    )(u0, u1, bt, qpad, wp, asn, adn, tflat, mask)


def _final_body(u0, u1, bt, qpad, wf1, bf1, wf2, bf2, out_o):
    g = (u0[...] + u1[...]) * (1.0 / NHEAD) + bt[0:1, :]
    e = jnp.where(g > 0, g, jnp.exp(g) - 1.0)
    hblk = jnp.concatenate([e, qpad[...]], axis=1)
    z = jnp.dot(hblk, wf1[...], preferred_element_type=jnp.float32)
    z = z + bf1[0:1, :]
    z = jnp.where(z > 0, z, jnp.exp(z) - 1.0)
    out_o[...] = jnp.dot(z, wf2[...],
                         preferred_element_type=jnp.float32) + bf2[0:1, :]


def _final(u0, u1, bt, qpad, wf1, bf1, wf2, bf2):
    return pl.pallas_call(
        _final_body,
        grid=(GRID,),
        in_specs=[_row_spec(NHID), _row_spec(NHID), _full_spec(8, NHID),
                  _row_spec(NHID), _full_spec(HP, 384), _full_spec(8, 384),
                  _full_spec(384, NHID), _full_spec(8, NHID)],
        out_specs=_row_spec(NHID),
        out_shape=jax.ShapeDtypeStruct((N, NHID), jnp.float32),
    )(u0, u1, bt, qpad, wf1, bf1, wf2, bf2)


_MESH = plsc.VectorSubcoreMesh(core_axis_name="c", subcore_axis_name="s",
                               num_cores=NC, num_subcores=NS)
_SC_PARAMS = pltpu.CompilerParams(use_tc_tiling_on_sc=False,
                                  needs_layout_passes=False)


def _zero16(ref, rows):
    def body(i, _):
        ref[i, :] = jnp.zeros((L,), jnp.float32)
        return 0
    lax.fori_loop(0, rows, body, 0)


def _pass1_body(src_h, dst_h, s1_h, ad_h, ex_h, den_h,
                src_v, dst_v, s1_v, ad_v, ex_v, zb, den_sp, sem):
    cid = lax.axis_index("c")
    sid = lax.axis_index("s")
    wid = sid * NC + cid

    _zero16(zb, ALN)
    pltpu.sync_copy(zb, den_sp.at[pl.ds(sid * ALN, ALN)])

    @pl.when(sid == 0)
    def _():
        pltpu.sync_copy(zb.at[pl.ds(0, TAIL)],
                        den_sp.at[pl.ds(NS * ALN, TAIL)])
    plsc.subcore_barrier()

    def chunk(c, _):
        base = wid * EPW + c * K
        pltpu.sync_copy(src_h.at[pl.ds(base, K)], src_v)
        pltpu.sync_copy(dst_h.at[pl.ds(base, K)], dst_v)
        pltpu.async_copy(s1_h.at[src_v], s1_v, sem).wait()
        pltpu.async_copy(ad_h.at[dst_v], ad_v, sem).wait()

        def edge(i, _):
            v = s1_v[i, :] + ad_v[i, :]
            v = jnp.where(v >= 0, v, v * 0.2)
            ex_v[i, :] = jnp.exp(v)
            return 0
        lax.fori_loop(0, K, edge, 0)

        pltpu.sync_copy(ex_v, den_sp.at[dst_v], add=True)
        pltpu.sync_copy(ex_v, ex_h.at[pl.ds(base, K)])
        return 0
    lax.fori_loop(0, NCHUNK, chunk, 0)

    plsc.subcore_barrier()
    pltpu.sync_copy(den_sp.at[pl.ds(sid * ALN, ALN)],
                    den_h.at[cid, pl.ds(sid * ALN, ALN)])

    @pl.when(sid == 0)
    def _():
        pltpu.sync_copy(den_sp.at[pl.ds(NS * ALN, TAIL)],
                        den_h.at[cid, pl.ds(NS * ALN, TAIL)])


def _pass1(src, dst, s1p, adp):
    return pl.kernel(
        _pass1_body,
        out_type=(jax.ShapeDtypeStruct((E, LP), jnp.float32),
                  jax.ShapeDtypeStruct((NC, N, LP), jnp.float32)),
        mesh=_MESH,
        scratch_types=[
            pltpu.VMEM((K,), jnp.int32),
            pltpu.VMEM((K,), jnp.int32),
            pltpu.VMEM((K, LP), jnp.float32),
            pltpu.VMEM((K, LP), jnp.float32),
            pltpu.VMEM((K, LP), jnp.float32),
            pltpu.VMEM((ALN, LP), jnp.float32),
            pltpu.VMEM_SHARED((N, LP), jnp.float32),
            pltpu.SemaphoreType.DMA,
        ],
        compiler_params=_SC_PARAMS,
    )(src, dst, s1p, adp)


_ZROWS = 48


def _pass2_body(src_h, dst_h, hw_h, ex_h, d0_h, d1_h, up_h,
                src_v, dst_v, rows_v, ex_v, d0_v, d1_v, msg_v, zb,
                out_sp, sem):
    cid = lax.axis_index("c")
    sid = lax.axis_index("s")
    wid = sid * NC + cid
    lane = lax.broadcasted_iota(jnp.int32, (L,), 0)

    def zrow(i, _):
        for j in range(8):
            zb[i, pl.ds(j * L, L)] = jnp.zeros((L,), jnp.float32)
        return 0
    lax.fori_loop(0, _ZROWS, zrow, 0)
    for kk in range(ALN // _ZROWS):
        pltpu.sync_copy(zb, out_sp.at[pl.ds(sid * ALN + kk * _ZROWS, _ZROWS)])

    @pl.when(sid == 0)
    def _():
        pltpu.sync_copy(zb.at[pl.ds(0, TAIL)],
                        out_sp.at[pl.ds(NS * ALN, TAIL)])
    plsc.subcore_barrier()

    def chunk(c, _):
        base = wid * EPW + c * K2
        pltpu.sync_copy(src_h.at[pl.ds(base, K2)], src_v)
        pltpu.sync_copy(dst_h.at[pl.ds(base, K2)], dst_v)
        pltpu.async_copy(hw_h.at[src_v], rows_v, sem).wait()
        pltpu.sync_copy(ex_h.at[pl.ds(base, K2)], ex_v)
        pltpu.async_copy(d0_h.at[dst_v], d0_v, sem).wait()
        pltpu.async_copy(d1_h.at[dst_v], d1_v, sem).wait()

        def edge(i, _):
            w = ex_v[i, :] / (d0_v[i, :] + d1_v[i, :] + 1e-16)
            acc = [jnp.zeros((L,), jnp.float32) for _ in range(8)]
            for h in range(NHEAD):
                wh = jnp.sum(jnp.where(lane == h, w, 0.0))
                for j in range(8):
                    acc[j] = acc[j] + wh * rows_v[i, pl.ds(h * NHID + j * L, L)]
            for j in range(8):
                msg_v[i, pl.ds(j * L, L)] = acc[j]
            return 0
        lax.fori_loop(0, K2, edge, 0)

        pltpu.sync_copy(msg_v, out_sp.at[dst_v], add=True)
        return 0
    lax.fori_loop(0, NCHUNK2, chunk, 0)

    plsc.subcore_barrier()
    pltpu.sync_copy(out_sp.at[pl.ds(sid * ALN, ALN)],
                    up_h.at[cid, pl.ds(sid * ALN, ALN)])

    @pl.when(sid == 0)
    def _():
        pltpu.sync_copy(out_sp.at[pl.ds(NS * ALN, TAIL)],
                        up_h.at[cid, pl.ds(NS * ALN, TAIL)])


def _pass2(src, dst, hw, ex, d0, d1):
    return pl.kernel(
        _pass2_body,
        out_type=jax.ShapeDtypeStruct((NC, N, NHID), jnp.float32),
        mesh=_MESH,
        scratch_types=[
            pltpu.VMEM((K2,), jnp.int32),
            pltpu.VMEM((K2,), jnp.int32),
            pltpu.VMEM((K2, HW), jnp.float32),
            pltpu.VMEM((K2, LP), jnp.float32),
            pltpu.VMEM((K2, LP), jnp.float32),
            pltpu.VMEM((K2, LP), jnp.float32),
            pltpu.VMEM((K2, NHID), jnp.float32),
            pltpu.VMEM((_ZROWS, NHID), jnp.float32),
            pltpu.VMEM_SHARED((N, NHID), jnp.float32),
            pltpu.SemaphoreType.DMA,
        ],
        compiler_params=_SC_PARAMS,
    )(src, dst, hw, ex, d0, d1)


def kernel(x, q_Y_sample, adj, t, num_steps, W_t1, b_t1, W_t2, b_t2, Wr,
           Wg0, as0, ad0, bg0, Wg1, as1, ad1, bg1, Wg2, as2, ad2, bg2,
           Wf1, bf1, Wf2, bf2):
    f32 = jnp.float32

    tv = t.astype(f32)
    half = NHID // 2
    emb = math.log(10000.0) / (half - 1)
    freqs = jnp.exp(jnp.arange(half, dtype=f32) * -emb)
    a = (tv * 4.0)[:, None] * freqs[None, :]
    pe_abs = jnp.concatenate([jnp.sin(a), jnp.cos(a)], axis=-1)
    pe_abs = jnp.zeros((8, NHID), f32).at[0:1].set(pe_abs)

    inv_freq = 1.0 / (10000.0 ** (jnp.arange(0.0, DIN, 2.0, dtype=f32) / DIN))
    si = tv[:, None] * inv_freq[None, :]
    pe_rel = jnp.concatenate([jnp.sin(si), jnp.cos(si)], axis=-1)
    pe_rel = jnp.zeros((8, HP), f32).at[0:1, :DREL].set(pe_rel)

    w1 = W_t1.astype(f32)
    b1 = jnp.zeros((8, 2 * NHID), f32).at[0].set(b_t1)
    w2 = W_t2.astype(f32)
    b2 = jnp.zeros((8, NHID), f32).at[0].set(b_t2)
    wr_p = jnp.zeros((HP, HW), f32).at[:DREL].set(Wr)

    t_abs, tflat = _prep(pe_abs, w1, b1, w2, b2, pe_rel, wr_p)

    rows768 = jnp.arange(HW) // NHID
    mask = (rows768[:, None] == jnp.arange(LP)[None, :]).astype(f32)

    def pad_w(w):
        return jnp.zeros((HP, HW), f32).at[:DIN].set(w)

    def pad_vec(v):
        return jnp.zeros((8, HW), f32).at[0].set(v.reshape(HW))

    qpad = jnp.zeros((N, NHID), f32).at[:, :NLABEL].set(q_Y_sample)
    hpad0 = jnp.concatenate([x.astype(f32), qpad], axis=1)

    src = adj[0].astype(jnp.int32)
    dst = adj[1].astype(jnp.int32)

    bts = [jnp.zeros((8, NHID), f32).at[0].set(b)[0:1] + t_abs[0:1]
           for b in (bg0, bg1, bg2)]
    bts = [jnp.concatenate([b, jnp.zeros((7, NHID), f32)], axis=0)
           for b in bts]

    layers = [(pad_w(Wg0), pad_vec(as0), pad_vec(ad0)),
              (pad_w(Wg1), pad_vec(as1), pad_vec(ad1)),
              (pad_w(Wg2), pad_vec(as2), pad_vec(ad2))]

    u0 = u1 = None
    for li, (wp, asn, adn) in enumerate(layers):
        if li == 0:
            hw, s1p, adp = _layer0(hpad0, wp, asn, adn, tflat, mask)
        else:
            hw, s1p, adp = _layern(u0, u1, bts[li - 1], qpad,
                                   wp, asn, adn, tflat, mask)
        ex, den = _pass1(src, dst, s1p, adp)
        up = _pass2(src, dst, hw, ex, den[0], den[1])
        u0, u1 = up[0], up[1]

    wf1 = jnp.zeros((HP, 384), f32).at[:FDIM, :2 * FDIM].set(Wf1)
    bf1p = jnp.zeros((8, 384), f32).at[0, :2 * FDIM].set(bf1)
    wf2 = jnp.zeros((384, NHID), f32).at[:2 * FDIM, :2 * NLABEL].set(Wf2)
    bf2p = jnp.zeros((8, NHID), f32).at[0, :2 * NLABEL].set(bf2)

    out = _final(u0, u1, bts[2], qpad, wf1, bf1p, wf2, bf2p)
    return out[:, :2 * NLABEL]


# speedup vs baseline: 41.2400x; 2.0485x over previous
import math

import jax
import jax.numpy as jnp
from jax import lax
from jax.experimental import pallas as pl
from jax.experimental.pallas import tpu as pltpu
from jax.experimental.pallas import tpu_sc as plsc

N = 10000
E = 320000
NFEAT = 128
NLABEL = 5
NHID = 128
NHEAD = 6
DIN = NFEAT + NLABEL
DREL = DIN + 1
FDIM = NHID + NLABEL
HW = NHEAD * NHID
HWH = HW // 2
CH = 64
HP = 256
LP = 16

NC, NS, L = 2, 16, 16
NWORK = NC * NS
EPW = E // NWORK
K = 80
NCHUNK = EPW // K
EPT = E // NS
K2 = 40
NCHUNK2 = EPT // K2
ALN = 624
TAIL = N - NS * ALN

BLK = 2000
GRID = N // BLK


def _prep_body(pe_abs, w1, b1, w2, b2, pe_rel, wr, t_abs_o, tflat_o):
    z = jnp.dot(pe_abs[...], w1[...], preferred_element_type=jnp.float32)
    z = z + b1[...]
    z = jnp.where(z > 0, z, jnp.exp(z) - 1.0)
    ta = jnp.dot(z, w2[...], preferred_element_type=jnp.float32) + b2[...]
    t_abs_o[...] = ta
    tflat_o[...] = jnp.dot(pe_rel[...], wr[...],
                           preferred_element_type=jnp.float32)


def _prep(pe_abs, w1, b1, w2, b2, pe_rel, wr):
    return pl.pallas_call(
        _prep_body,
        out_shape=(jax.ShapeDtypeStruct((8, NHID), jnp.float32),
                   jax.ShapeDtypeStruct((8, HW), jnp.float32)),
    )(pe_abs, w1, b1, w2, b2, pe_rel, wr)


def _matmul_tail(hblk, wlo, whi, aslo, ashi, adlo, adhi, tflo, tfhi, mask64,
                 lo_o, hi_o, s1_o, ad_o):
    lo = jnp.dot(hblk, wlo[...], preferred_element_type=jnp.float32)
    hi = jnp.dot(hblk, whi[...], preferred_element_type=jnp.float32)
    lo_o[...] = lo
    hi_o[...] = hi
    svlo = aslo[0:1, :] + tflo[0:1, :]
    svhi = ashi[0:1, :] + tfhi[0:1, :]
    s1_o[...] = (jnp.dot(lo * svlo, mask64[...],
                         preferred_element_type=jnp.float32) +
                 jnp.dot(hi * svhi, mask64[...],
                         preferred_element_type=jnp.float32))
    ad_o[...] = (jnp.dot(lo * adlo[0:1, :], mask64[...],
                         preferred_element_type=jnp.float32) +
                 jnp.dot(hi * adhi[0:1, :], mask64[...],
                         preferred_element_type=jnp.float32))


def _layer0_body(hpad, wlo, whi, aslo, ashi, adlo, adhi, tflo, tfhi, mask64,
                 lo_o, hi_o, s1_o, ad_o):
    _matmul_tail(hpad[...], wlo, whi, aslo, ashi, adlo, adhi, tflo, tfhi,
                 mask64, lo_o, hi_o, s1_o, ad_o)


def _layern_body(u0, u1, bt, qpad, wlo, whi, aslo, ashi, adlo, adhi,
                 tflo, tfhi, mask64, lo_o, hi_o, s1_o, ad_o):
    g = jnp.concatenate([u0[...], u1[...]], axis=1) * (1.0 / NHEAD)
    g = g + bt[0:1, :]
    e = jnp.where(g > 0, g, jnp.exp(g) - 1.0)
    hblk = jnp.concatenate([e, qpad[...]], axis=1)
    _matmul_tail(hblk, wlo, whi, aslo, ashi, adlo, adhi, tflo, tfhi,
                 mask64, lo_o, hi_o, s1_o, ad_o)


def _row_spec(w):
    return pl.BlockSpec((BLK, w), lambda i: (i, 0))


def _full_spec(r, c):
    return pl.BlockSpec((r, c), lambda i: (0, 0))


_LAYER_OUT = (jax.ShapeDtypeStruct((N, HWH), jnp.float32),
              jax.ShapeDtypeStruct((N, HWH), jnp.float32),
              jax.ShapeDtypeStruct((N, LP), jnp.float32),
              jax.ShapeDtypeStruct((N, LP), jnp.float32))
_LAYER_OUT_SPECS = (_row_spec(HWH), _row_spec(HWH),
                    _row_spec(LP), _row_spec(LP))
_WSPECS = [_full_spec(HP, HWH), _full_spec(HP, HWH), _full_spec(8, HWH),
           _full_spec(8, HWH), _full_spec(8, HWH), _full_spec(8, HWH),
           _full_spec(8, HWH), _full_spec(8, HWH), _full_spec(HWH, LP)]


def _layer0(hpad, *ws):
    return pl.pallas_call(
        _layer0_body,
        grid=(GRID,),
        in_specs=[_row_spec(HP)] + _WSPECS,
        out_specs=_LAYER_OUT_SPECS,
        out_shape=_LAYER_OUT,
    )(hpad, *ws)


def _layern(u0, u1, bt, qpad, *ws):
    return pl.pallas_call(
        _layern_body,
        grid=(GRID,),
        in_specs=[_row_spec(CH), _row_spec(CH), _full_spec(8, NHID),
                  _row_spec(NHID)] + _WSPECS,
        out_specs=_LAYER_OUT_SPECS,
        out_shape=_LAYER_OUT,
    )(u0, u1, bt, qpad, *ws)


def _final_body(u0, u1, bt, qpad, wf1, bf1, wf2, bf2, out_o):
    g = jnp.concatenate([u0[...], u1[...]], axis=1) * (1.0 / NHEAD)
    g = g + bt[0:1, :]
    e = jnp.where(g > 0, g, jnp.exp(g) - 1.0)
    hblk = jnp.concatenate([e, qpad[...]], axis=1)
    z = jnp.dot(hblk, wf1[...], preferred_element_type=jnp.float32)
    z = z + bf1[0:1, :]
    z = jnp.where(z > 0, z, jnp.exp(z) - 1.0)
    out_o[...] = jnp.dot(z, wf2[...],
                         preferred_element_type=jnp.float32) + bf2[0:1, :]


def _final(u0, u1, bt, qpad, wf1, bf1, wf2, bf2):
    return pl.pallas_call(
        _final_body,
        grid=(GRID,),
        in_specs=[_row_spec(CH), _row_spec(CH), _full_spec(8, NHID),
                  _row_spec(NHID), _full_spec(HP, 384), _full_spec(8, 384),
                  _full_spec(384, NHID), _full_spec(8, NHID)],
        out_specs=_row_spec(NHID),
        out_shape=jax.ShapeDtypeStruct((N, NHID), jnp.float32),
    )(u0, u1, bt, qpad, wf1, bf1, wf2, bf2)


_MESH = plsc.VectorSubcoreMesh(core_axis_name="c", subcore_axis_name="s",
                               num_cores=NC, num_subcores=NS)
_SC_PARAMS = pltpu.CompilerParams(use_tc_tiling_on_sc=False,
                                  needs_layout_passes=False)


def _pass1_body(src_h, dst_h, s1_h, ad_h, ex_h, den_h,
                src_pf, dst_pf, s1_a, s1_b, ad_a, ad_b, ex_v, zb, den_sp,
                sem_a, sem_b):
    cid = lax.axis_index("c")
    sid = lax.axis_index("s")
    wid = sid * NC + cid
    s1_v = (s1_a, s1_b)
    ad_v = (ad_a, ad_b)
    sem = (sem_a, sem_b)

    def zrow(i, _):
        zb[i, :] = jnp.zeros((L,), jnp.float32)
        return 0
    lax.fori_loop(0, ALN, zrow, 0)
    pltpu.sync_copy(zb, den_sp.at[pl.ds(sid * ALN, ALN)])

    @pl.when(sid == 0)
    def _():
        pltpu.sync_copy(zb.at[pl.ds(0, TAIL)],
                        den_sp.at[pl.ds(NS * ALN, TAIL)])

    pltpu.sync_copy(src_h.at[wid], src_pf)
    pltpu.sync_copy(dst_h.at[wid], dst_pf)
    plsc.subcore_barrier()

    def fire(c, b):
        pltpu.async_copy(s1_h.at[src_pf.at[c]], s1_v[b], sem[b])
        pltpu.async_copy(ad_h.at[dst_pf.at[c]], ad_v[b], sem[b])

    def drain(c, b):
        pltpu.make_async_copy(s1_h.at[src_pf.at[c]], s1_v[b], sem[b]).wait()
        pltpu.make_async_copy(ad_h.at[dst_pf.at[c]], ad_v[b], sem[b]).wait()

    def work(c, b):
        def edge(i, _):
            v = s1_v[b][i, :] + ad_v[b][i, :]
            v = jnp.where(v >= 0, v, v * 0.2)
            ex_v[i, :] = jnp.exp(v)
            return 0
        lax.fori_loop(0, K, edge, 0)
        pltpu.sync_copy(ex_v, den_sp.at[dst_pf.at[c]], add=True)
        pltpu.sync_copy(ex_v, ex_h.at[pl.ds(wid * EPW + c * K, K)])

    fire(0, 0)

    def pair(g, _):
        c0 = 2 * g
        fire(c0 + 1, 1)
        drain(c0, 0)
        work(c0, 0)
        fire(c0 + 2, 0)
        drain(c0 + 1, 1)
        work(c0 + 1, 1)
        return 0
    lax.fori_loop(0, (NCHUNK - 1) // 2, pair, 0)
    drain(NCHUNK - 1, 0)
    work(NCHUNK - 1, 0)

    plsc.subcore_barrier()
    pltpu.sync_copy(den_sp.at[pl.ds(sid * ALN, ALN)],
                    den_h.at[cid, pl.ds(sid * ALN, ALN)])

    @pl.when(sid == 0)
    def _():
        pltpu.sync_copy(den_sp.at[pl.ds(NS * ALN, TAIL)],
                        den_h.at[cid, pl.ds(NS * ALN, TAIL)])


def _pass1(src3, dst3, s1p, adp):
    return pl.kernel(
        _pass1_body,
        out_type=(jax.ShapeDtypeStruct((E, LP), jnp.float32),
                  jax.ShapeDtypeStruct((NC, N, LP), jnp.float32)),
        mesh=_MESH,
        scratch_types=[
            pltpu.VMEM((NCHUNK, K), jnp.int32),
            pltpu.VMEM((NCHUNK, K), jnp.int32),
            pltpu.VMEM((K, LP), jnp.float32),
            pltpu.VMEM((K, LP), jnp.float32),
            pltpu.VMEM((K, LP), jnp.float32),
            pltpu.VMEM((K, LP), jnp.float32),
            pltpu.VMEM((K, LP), jnp.float32),
            pltpu.VMEM((ALN, LP), jnp.float32),
            pltpu.VMEM_SHARED((N, LP), jnp.float32),
            pltpu.SemaphoreType.DMA,
            pltpu.SemaphoreType.DMA,
        ],
        compiler_params=_SC_PARAMS,
    )(src3, dst3, s1p, adp)


_ZROWS = 48


def _pass2_body(src_h, dst_h, lo_h, hi_h, ex_h, d0_h, d1_h, up_h,
                src_pf, dst_pf, rows_a, rows_b, ex_a, ex_b,
                d0_a, d0_b, d1_a, d1_b, msg_v, zb, out_sp, sem_a, sem_b):
    cid = lax.axis_index("c")
    sid = lax.axis_index("s")
    lane = lax.broadcasted_iota(jnp.int32, (L,), 0)
    rows_v = (rows_a, rows_b)
    ex_v = (ex_a, ex_b)
    d0_v = (d0_a, d0_b)
    d1_v = (d1_a, d1_b)
    sem = (sem_a, sem_b)

    def zrow(i, _):
        for j in range(CH // L):
            zb[i, pl.ds(j * L, L)] = jnp.zeros((L,), jnp.float32)
        return 0
    lax.fori_loop(0, _ZROWS, zrow, 0)
    for kk in range(ALN // _ZROWS):
        pltpu.sync_copy(zb, out_sp.at[pl.ds(sid * ALN + kk * _ZROWS, _ZROWS)])

    @pl.when(sid == 0)
    def _():
        pltpu.sync_copy(zb.at[pl.ds(0, TAIL)],
                        out_sp.at[pl.ds(NS * ALN, TAIL)])

    pltpu.sync_copy(src_h.at[sid], src_pf)
    pltpu.sync_copy(dst_h.at[sid], dst_pf)
    plsc.subcore_barrier()

    def sweep(tab_h):
        def fire(c, b):
            pltpu.async_copy(tab_h.at[src_pf.at[c]], rows_v[b], sem[b])
            pltpu.async_copy(ex_h.at[pl.ds(sid * EPT + c * K2, K2)],
                             ex_v[b], sem[b])
            pltpu.async_copy(d0_h.at[dst_pf.at[c]], d0_v[b], sem[b])
            pltpu.async_copy(d1_h.at[dst_pf.at[c]], d1_v[b], sem[b])

        def drain(c, b):
            pltpu.make_async_copy(tab_h.at[src_pf.at[c]], rows_v[b],
                                  sem[b]).wait()
            pltpu.make_async_copy(ex_h.at[pl.ds(sid * EPT + c * K2, K2)],
                                  ex_v[b], sem[b]).wait()
            pltpu.make_async_copy(d0_h.at[dst_pf.at[c]], d0_v[b],
                                  sem[b]).wait()
            pltpu.make_async_copy(d1_h.at[dst_pf.at[c]], d1_v[b],
                                  sem[b]).wait()

        def work(c, b):
            def edge(i, _):
                w = ex_v[b][i, :] / (d0_v[b][i, :] + d1_v[b][i, :] + 1e-16)
                acc = [None] * (CH // L)
                for h in range(NHEAD):
                    wh = jnp.sum(jnp.where(lane == h, w, 0.0))
                    for j in range(CH // L):
                        r = wh * rows_v[b][i, pl.ds(h * CH + j * L, L)]
                        acc[j] = r if h == 0 else acc[j] + r
                for j in range(CH // L):
                    msg_v[i, pl.ds(j * L, L)] = acc[j]
                return 0
            lax.fori_loop(0, K2, edge, 0)
            pltpu.sync_copy(msg_v, out_sp.at[dst_pf.at[c]], add=True)

        fire(0, 0)

        def pair(g, _):
            c0 = 2 * g
            fire(c0 + 1, 1)
            drain(c0, 0)
            work(c0, 0)
            fire(c0 + 2, 0)
            drain(c0 + 1, 1)
            work(c0 + 1, 1)
            return 0
        lax.fori_loop(0, (NCHUNK2 - 2) // 2, pair, 0)
        c0 = NCHUNK2 - 2
        fire(c0 + 1, 1)
        drain(c0, 0)
        work(c0, 0)
        drain(c0 + 1, 1)
        work(c0 + 1, 1)

    @pl.when(cid == 0)
    def _():
        sweep(lo_h)

    @pl.when(cid == 1)
    def _():
        sweep(hi_h)

    plsc.subcore_barrier()
    pltpu.sync_copy(out_sp.at[pl.ds(sid * ALN, ALN)],
                    up_h.at[cid, pl.ds(sid * ALN, ALN)])

    @pl.when(sid == 0)
    def _():
        pltpu.sync_copy(out_sp.at[pl.ds(NS * ALN, TAIL)],
                        up_h.at[cid, pl.ds(NS * ALN, TAIL)])


def _pass2(src3, dst3, hwlo, hwhi, ex, d0, d1):
    return pl.kernel(
        _pass2_body,
        out_type=jax.ShapeDtypeStruct((NC, N, CH), jnp.float32),
        mesh=_MESH,
        scratch_types=[
            pltpu.VMEM((NCHUNK2, K2), jnp.int32),
            pltpu.VMEM((NCHUNK2, K2), jnp.int32),
            pltpu.VMEM((K2, HWH), jnp.float32),
            pltpu.VMEM((K2, HWH), jnp.float32),
            pltpu.VMEM((K2, LP), jnp.float32),
            pltpu.VMEM((K2, LP), jnp.float32),
            pltpu.VMEM((K2, LP), jnp.float32),
            pltpu.VMEM((K2, LP), jnp.float32),
            pltpu.VMEM((K2, LP), jnp.float32),
            pltpu.VMEM((K2, LP), jnp.float32),
            pltpu.VMEM((K2, CH), jnp.float32),
            pltpu.VMEM((_ZROWS, CH), jnp.float32),
            pltpu.VMEM_SHARED((N, CH), jnp.float32),
            pltpu.SemaphoreType.DMA,
            pltpu.SemaphoreType.DMA,
        ],
        compiler_params=_SC_PARAMS,
    )(src3, dst3, hwlo, hwhi, ex, d0, d1)


def kernel(x, q_Y_sample, adj, t, num_steps, W_t1, b_t1, W_t2, b_t2, Wr,
           Wg0, as0, ad0, bg0, Wg1, as1, ad1, bg1, Wg2, as2, ad2, bg2,
           Wf1, bf1, Wf2, bf2):
    f32 = jnp.float32

    tv = t.astype(f32)
    half = NHID // 2
    emb = math.log(10000.0) / (half - 1)
    freqs = jnp.exp(jnp.arange(half, dtype=f32) * -emb)
    a = (tv * 4.0)[:, None] * freqs[None, :]
    pe_abs = jnp.concatenate([jnp.sin(a), jnp.cos(a)], axis=-1)
    pe_abs = jnp.zeros((8, NHID), f32).at[0:1].set(pe_abs)

    inv_freq = 1.0 / (10000.0 ** (jnp.arange(0.0, DIN, 2.0, dtype=f32) / DIN))
    si = tv[:, None] * inv_freq[None, :]
    pe_rel = jnp.concatenate([jnp.sin(si), jnp.cos(si)], axis=-1)
    pe_rel = jnp.zeros((8, HP), f32).at[0:1, :DREL].set(pe_rel)

    w1 = W_t1.astype(f32)
    b1 = jnp.zeros((8, 2 * NHID), f32).at[0].set(b_t1)
    w2 = W_t2.astype(f32)
    b2 = jnp.zeros((8, NHID), f32).at[0].set(b_t2)
    wr_p = jnp.zeros((HP, HW), f32).at[:DREL].set(Wr)

    t_abs, tflat = _prep(pe_abs, w1, b1, w2, b2, pe_rel, wr_p)

    cols_lo = (jnp.arange(HWH) // CH) * NHID + jnp.arange(HWH) % CH
    cols_hi = cols_lo + CH
    rows384 = jnp.arange(HWH) // CH
    mask64 = (rows384[:, None] == jnp.arange(LP)[None, :]).astype(f32)

    tflo = tflat[:, cols_lo]
    tfhi = tflat[:, cols_hi]

    def mk_ws(wg, a_s, a_d):
        wp = jnp.zeros((HP, HW), f32).at[:DIN].set(wg)
        asf = a_s.reshape(HW)
        adf = a_d.reshape(HW)

        def row8(v):
            return jnp.zeros((8, HWH), f32).at[0].set(v)
        return (wp[:, cols_lo], wp[:, cols_hi],
                row8(asf[cols_lo]), row8(asf[cols_hi]),
                row8(adf[cols_lo]), row8(adf[cols_hi]),
                tflo, tfhi, mask64)

    qpad = jnp.zeros((N, NHID), f32).at[:, :NLABEL].set(q_Y_sample)
    hpad0 = jnp.concatenate([x.astype(f32), qpad], axis=1)

    src = adj[0].astype(jnp.int32)
    dst = adj[1].astype(jnp.int32)
    src3 = src.reshape(NWORK, NCHUNK, K)
    dst3 = dst.reshape(NWORK, NCHUNK, K)
    src3b = src.reshape(NS, NCHUNK2, K2)
    dst3b = dst.reshape(NS, NCHUNK2, K2)

    bts = [jnp.zeros((8, NHID), f32).at[0].set(b)[0:1] + t_abs[0:1]
           for b in (bg0, bg1, bg2)]
    bts = [jnp.concatenate([b, jnp.zeros((7, NHID), f32)], axis=0)
           for b in bts]

    layers = [mk_ws(Wg0, as0, ad0), mk_ws(Wg1, as1, ad1), mk_ws(Wg2, as2, ad2)]

    u0 = u1 = None
    for li, ws in enumerate(layers):
        if li == 0:
            hwlo, hwhi, s1p, adp = _layer0(hpad0, *ws)
        else:
            hwlo, hwhi, s1p, adp = _layern(u0, u1, bts[li - 1], qpad, *ws)
        ex, den = _pass1(src3, dst3, s1p, adp)
        up = _pass2(src3b, dst3b, hwlo, hwhi, ex, den[0], den[1])
        u0, u1 = up[0], up[1]

    wf1 = jnp.zeros((HP, 384), f32).at[:FDIM, :2 * FDIM].set(Wf1)
    bf1p = jnp.zeros((8, 384), f32).at[0, :2 * FDIM].set(bf1)
    wf2 = jnp.zeros((384, NHID), f32).at[:2 * FDIM, :2 * NLABEL].set(Wf2)
    bf2p = jnp.zeros((8, NHID), f32).at[0, :2 * NLABEL].set(bf2)

    out = _final(u0, u1, bts[2], qpad, wf1, bf1p, wf2, bf2p)
    return out[:, :2 * NLABEL]


# speedup vs baseline: 47.3318x; 1.1477x over previous
import math

import jax
import jax.numpy as jnp
from jax import lax
from jax.experimental import pallas as pl
from jax.experimental.pallas import tpu as pltpu
from jax.experimental.pallas import tpu_sc as plsc

N = 10000
E = 320000
NFEAT = 128
NLABEL = 5
NHID = 128
NHEAD = 6
DIN = NFEAT + NLABEL
DREL = DIN + 1
FDIM = NHID + NLABEL
HW = NHEAD * NHID
HWH = HW // 2
CH = 64
HP = 256
LP = 16

NC, NS, L = 2, 16, 16
NWORK = NC * NS
EPW = E // NWORK
K = 80
NCHUNK = EPW // K
EPT = E // NS
K2 = 40
NCHUNK2 = EPT // K2
ALN = 624
TAIL = N - NS * ALN

BLK = 2000
GRID = N // BLK


def _prep_body(pe_abs, w1, b1, w2, b2, pe_rel, wr, t_abs_o, tflat_o):
    z = jnp.dot(pe_abs[...], w1[...], preferred_element_type=jnp.float32)
    z = z + b1[...]
    z = jnp.where(z > 0, z, jnp.exp(z) - 1.0)
    ta = jnp.dot(z, w2[...], preferred_element_type=jnp.float32) + b2[...]
    t_abs_o[...] = ta
    tflat_o[...] = jnp.dot(pe_rel[...], wr[...],
                           preferred_element_type=jnp.float32)


def _prep(pe_abs, w1, b1, w2, b2, pe_rel, wr):
    return pl.pallas_call(
        _prep_body,
        out_shape=(jax.ShapeDtypeStruct((8, NHID), jnp.float32),
                   jax.ShapeDtypeStruct((8, HW), jnp.float32)),
    )(pe_abs, w1, b1, w2, b2, pe_rel, wr)


def _matmul_tail(hblk, wlo, whi, aslo, ashi, adlo, adhi, tflo, tfhi, mask64,
                 lo_o, hi_o, s1_o, ad_o):
    lo = jnp.dot(hblk, wlo[...], preferred_element_type=jnp.float32)
    hi = jnp.dot(hblk, whi[...], preferred_element_type=jnp.float32)
    lo_o[...] = lo
    hi_o[...] = hi
    svlo = aslo[0:1, :] + tflo[0:1, :]
    svhi = ashi[0:1, :] + tfhi[0:1, :]
    s1_o[...] = (jnp.dot(lo * svlo, mask64[...],
                         preferred_element_type=jnp.float32) +
                 jnp.dot(hi * svhi, mask64[...],
                         preferred_element_type=jnp.float32))
    ad_o[...] = (jnp.dot(lo * adlo[0:1, :], mask64[...],
                         preferred_element_type=jnp.float32) +
                 jnp.dot(hi * adhi[0:1, :], mask64[...],
                         preferred_element_type=jnp.float32))


def _layer0_body(hpad, wlo, whi, aslo, ashi, adlo, adhi, tflo, tfhi, mask64,
                 lo_o, hi_o, s1_o, ad_o):
    _matmul_tail(hpad[...], wlo, whi, aslo, ashi, adlo, adhi, tflo, tfhi,
                 mask64, lo_o, hi_o, s1_o, ad_o)


def _layern_body(u0, u1, bt, qpad, wlo, whi, aslo, ashi, adlo, adhi,
                 tflo, tfhi, mask64, lo_o, hi_o, s1_o, ad_o):
    g = jnp.concatenate([u0[...], u1[...]], axis=1) * (1.0 / NHEAD)
    g = g + bt[0:1, :]
    e = jnp.where(g > 0, g, jnp.exp(g) - 1.0)
    hblk = jnp.concatenate([e, qpad[...]], axis=1)
    _matmul_tail(hblk, wlo, whi, aslo, ashi, adlo, adhi, tflo, tfhi,
                 mask64, lo_o, hi_o, s1_o, ad_o)


def _row_spec(w):
    return pl.BlockSpec((BLK, w), lambda i: (i, 0))


def _full_spec(r, c):
    return pl.BlockSpec((r, c), lambda i: (0, 0))


_LAYER_OUT = (jax.ShapeDtypeStruct((N, HWH), jnp.float32),
              jax.ShapeDtypeStruct((N, HWH), jnp.float32),
              jax.ShapeDtypeStruct((N, LP), jnp.float32),
              jax.ShapeDtypeStruct((N, LP), jnp.float32))
_LAYER_OUT_SPECS = (_row_spec(HWH), _row_spec(HWH),
                    _row_spec(LP), _row_spec(LP))
_WSPECS = [_full_spec(HP, HWH), _full_spec(HP, HWH), _full_spec(8, HWH),
           _full_spec(8, HWH), _full_spec(8, HWH), _full_spec(8, HWH),
           _full_spec(8, HWH), _full_spec(8, HWH), _full_spec(HWH, LP)]


def _layer0(hpad, *ws):
    return pl.pallas_call(
        _layer0_body,
        grid=(GRID,),
        in_specs=[_row_spec(HP)] + _WSPECS,
        out_specs=_LAYER_OUT_SPECS,
        out_shape=_LAYER_OUT,
    )(hpad, *ws)


def _layern(u0, u1, bt, qpad, *ws):
    return pl.pallas_call(
        _layern_body,
        grid=(GRID,),
        in_specs=[_row_spec(CH), _row_spec(CH), _full_spec(8, NHID),
                  _row_spec(NHID)] + _WSPECS,
        out_specs=_LAYER_OUT_SPECS,
        out_shape=_LAYER_OUT,
    )(u0, u1, bt, qpad, *ws)


def _final_body(u0, u1, bt, qpad, wf1, bf1, wf2, bf2, out_o):
    g = jnp.concatenate([u0[...], u1[...]], axis=1) * (1.0 / NHEAD)
    g = g + bt[0:1, :]
    e = jnp.where(g > 0, g, jnp.exp(g) - 1.0)
    hblk = jnp.concatenate([e, qpad[...]], axis=1)
    z = jnp.dot(hblk, wf1[...], preferred_element_type=jnp.float32)
    z = z + bf1[0:1, :]
    z = jnp.where(z > 0, z, jnp.exp(z) - 1.0)
    out_o[...] = jnp.dot(z, wf2[...],
                         preferred_element_type=jnp.float32) + bf2[0:1, :]


def _final(u0, u1, bt, qpad, wf1, bf1, wf2, bf2):
    return pl.pallas_call(
        _final_body,
        grid=(GRID,),
        in_specs=[_row_spec(CH), _row_spec(CH), _full_spec(8, NHID),
                  _row_spec(NHID), _full_spec(HP, 384), _full_spec(8, 384),
                  _full_spec(384, NHID), _full_spec(8, NHID)],
        out_specs=_row_spec(NHID),
        out_shape=jax.ShapeDtypeStruct((N, NHID), jnp.float32),
    )(u0, u1, bt, qpad, wf1, bf1, wf2, bf2)


_MESH = plsc.VectorSubcoreMesh(core_axis_name="c", subcore_axis_name="s",
                               num_cores=NC, num_subcores=NS)
_SC_PARAMS = pltpu.CompilerParams(use_tc_tiling_on_sc=False,
                                  needs_layout_passes=False)


def _pass1_body(src_h, dst_h, s1_h, ad_h, ex_h, den_h,
                src_pf, dst_pf, s1_a, s1_b, ad_a, ad_b, ex_v, zb, den_sp,
                sem_a, sem_b):
    cid = lax.axis_index("c")
    sid = lax.axis_index("s")
    wid = sid * NC + cid
    s1_v = (s1_a, s1_b)
    ad_v = (ad_a, ad_b)
    sem = (sem_a, sem_b)

    def zrow(i, _):
        zb[i, :] = jnp.zeros((L,), jnp.float32)
        return 0
    lax.fori_loop(0, ALN, zrow, 0)
    pltpu.sync_copy(zb, den_sp.at[pl.ds(sid * ALN, ALN)])

    @pl.when(sid == 0)
    def _():
        pltpu.sync_copy(zb.at[pl.ds(0, TAIL)],
                        den_sp.at[pl.ds(NS * ALN, TAIL)])

    pltpu.sync_copy(src_h.at[wid], src_pf)
    pltpu.sync_copy(dst_h.at[wid], dst_pf)
    plsc.subcore_barrier()

    def fire(c, b):
        pltpu.async_copy(s1_h.at[src_pf.at[c]], s1_v[b], sem[b])
        pltpu.async_copy(ad_h.at[dst_pf.at[c]], ad_v[b], sem[b])

    def drain(c, b):
        pltpu.make_async_copy(s1_h.at[src_pf.at[c]], s1_v[b], sem[b]).wait()
        pltpu.make_async_copy(ad_h.at[dst_pf.at[c]], ad_v[b], sem[b]).wait()

    def work(c, b):
        def edge(i, _):
            v = s1_v[b][i, :] + ad_v[b][i, :]
            v = jnp.where(v >= 0, v, v * 0.2)
            ex_v[i, :] = jnp.exp(v)
            return 0
        lax.fori_loop(0, K, edge, 0)
        pltpu.sync_copy(ex_v, den_sp.at[dst_pf.at[c]], add=True)
        pltpu.sync_copy(ex_v, ex_h.at[pl.ds(wid * EPW + c * K, K)])

    fire(0, 0)

    def pair(g, _):
        c0 = 2 * g
        fire(c0 + 1, 1)
        drain(c0, 0)
        work(c0, 0)
        fire(c0 + 2, 0)
        drain(c0 + 1, 1)
        work(c0 + 1, 1)
        return 0
    lax.fori_loop(0, (NCHUNK - 1) // 2, pair, 0)
    drain(NCHUNK - 1, 0)
    work(NCHUNK - 1, 0)

    plsc.subcore_barrier()
    pltpu.sync_copy(den_sp.at[pl.ds(sid * ALN, ALN)],
                    den_h.at[cid, pl.ds(sid * ALN, ALN)])

    @pl.when(sid == 0)
    def _():
        pltpu.sync_copy(den_sp.at[pl.ds(NS * ALN, TAIL)],
                        den_h.at[cid, pl.ds(NS * ALN, TAIL)])


def _pass1(src3, dst3, s1p, adp):
    return pl.kernel(
        _pass1_body,
        out_type=(jax.ShapeDtypeStruct((E, LP), jnp.float32),
                  jax.ShapeDtypeStruct((NC, N, LP), jnp.float32)),
        mesh=_MESH,
        scratch_types=[
            pltpu.VMEM((NCHUNK, K), jnp.int32),
            pltpu.VMEM((NCHUNK, K), jnp.int32),
            pltpu.VMEM((K, LP), jnp.float32),
            pltpu.VMEM((K, LP), jnp.float32),
            pltpu.VMEM((K, LP), jnp.float32),
            pltpu.VMEM((K, LP), jnp.float32),
            pltpu.VMEM((K, LP), jnp.float32),
            pltpu.VMEM((ALN, LP), jnp.float32),
            pltpu.VMEM_SHARED((N, LP), jnp.float32),
            pltpu.SemaphoreType.DMA,
            pltpu.SemaphoreType.DMA,
        ],
        compiler_params=_SC_PARAMS,
    )(src3, dst3, s1p, adp)


_ZROWS = 48


def _pass2_body(src_h, dst_h, lo_h, hi_h, ex_h, d0_h, d1_h, up_h,
                src_pf, dst_pf, rows_a, rows_b, ex_a, ex_b,
                d0_a, d0_b, d1_a, d1_b, msg_v, zb, out_sp, sem_a, sem_b):
    cid = lax.axis_index("c")
    sid = lax.axis_index("s")
    lane = lax.broadcasted_iota(jnp.int32, (L,), 0)
    rows_v = (rows_a, rows_b)
    ex_v = (ex_a, ex_b)
    d0_v = (d0_a, d0_b)
    d1_v = (d1_a, d1_b)
    sem = (sem_a, sem_b)

    def zrow(i, _):
        for j in range(CH // L):
            zb[i, pl.ds(j * L, L)] = jnp.zeros((L,), jnp.float32)
        return 0
    lax.fori_loop(0, _ZROWS, zrow, 0)
    for kk in range(ALN // _ZROWS):
        pltpu.sync_copy(zb, out_sp.at[pl.ds(sid * ALN + kk * _ZROWS, _ZROWS)])

    @pl.when(sid == 0)
    def _():
        pltpu.sync_copy(zb.at[pl.ds(0, TAIL)],
                        out_sp.at[pl.ds(NS * ALN, TAIL)])

    pltpu.sync_copy(src_h.at[sid], src_pf)
    pltpu.sync_copy(dst_h.at[sid], dst_pf)
    plsc.subcore_barrier()

    def sweep(tab_h):
        def fire(c, b):
            pltpu.async_copy(tab_h.at[src_pf.at[c]], rows_v[b], sem[b])
            pltpu.async_copy(ex_h.at[pl.ds(sid * EPT + c * K2, K2)],
                             ex_v[b], sem[b])
            pltpu.async_copy(d0_h.at[dst_pf.at[c]], d0_v[b], sem[b])
            pltpu.async_copy(d1_h.at[dst_pf.at[c]], d1_v[b], sem[b])

        def drain(c, b):
            pltpu.make_async_copy(tab_h.at[src_pf.at[c]], rows_v[b],
                                  sem[b]).wait()
            pltpu.make_async_copy(ex_h.at[pl.ds(sid * EPT + c * K2, K2)],
                                  ex_v[b], sem[b]).wait()
            pltpu.make_async_copy(d0_h.at[dst_pf.at[c]], d0_v[b],
                                  sem[b]).wait()
            pltpu.make_async_copy(d1_h.at[dst_pf.at[c]], d1_v[b],
                                  sem[b]).wait()

        def work(c, b):
            def edge(i, _):
                w = ex_v[b][i, :] / (d0_v[b][i, :] + d1_v[b][i, :] + 1e-16)
                acc = [None] * (CH // L)
                for h in range(NHEAD):
                    wh = lax.gather(
                        w, (lane * 0 + h)[:, None],
                        lax.GatherDimensionNumbers(
                            offset_dims=(), collapsed_slice_dims=(0,),
                            start_index_map=(0,)),
                        (1,),
                        mode=lax.GatherScatterMode.PROMISE_IN_BOUNDS)
                    for j in range(CH // L):
                        r = wh * rows_v[b][i, pl.ds(h * CH + j * L, L)]
                        acc[j] = r if h == 0 else acc[j] + r
                for j in range(CH // L):
                    msg_v[i, pl.ds(j * L, L)] = acc[j]
                return 0
            lax.fori_loop(0, K2, edge, 0)
            pltpu.sync_copy(msg_v, out_sp.at[dst_pf.at[c]], add=True)

        fire(0, 0)

        def pair(g, _):
            c0 = 2 * g
            fire(c0 + 1, 1)
            drain(c0, 0)
            work(c0, 0)
            fire(c0 + 2, 0)
            drain(c0 + 1, 1)
            work(c0 + 1, 1)
            return 0
        lax.fori_loop(0, (NCHUNK2 - 2) // 2, pair, 0)
        c0 = NCHUNK2 - 2
        fire(c0 + 1, 1)
        drain(c0, 0)
        work(c0, 0)
        drain(c0 + 1, 1)
        work(c0 + 1, 1)

    @pl.when(cid == 0)
    def _():
        sweep(lo_h)

    @pl.when(cid == 1)
    def _():
        sweep(hi_h)

    plsc.subcore_barrier()
    pltpu.sync_copy(out_sp.at[pl.ds(sid * ALN, ALN)],
                    up_h.at[cid, pl.ds(sid * ALN, ALN)])

    @pl.when(sid == 0)
    def _():
        pltpu.sync_copy(out_sp.at[pl.ds(NS * ALN, TAIL)],
                        up_h.at[cid, pl.ds(NS * ALN, TAIL)])


def _pass2(src3, dst3, hwlo, hwhi, ex, d0, d1):
    return pl.kernel(
        _pass2_body,
        out_type=jax.ShapeDtypeStruct((NC, N, CH), jnp.float32),
        mesh=_MESH,
        scratch_types=[
            pltpu.VMEM((NCHUNK2, K2), jnp.int32),
            pltpu.VMEM((NCHUNK2, K2), jnp.int32),
            pltpu.VMEM((K2, HWH), jnp.float32),
            pltpu.VMEM((K2, HWH), jnp.float32),
            pltpu.VMEM((K2, LP), jnp.float32),
            pltpu.VMEM((K2, LP), jnp.float32),
            pltpu.VMEM((K2, LP), jnp.float32),
            pltpu.VMEM((K2, LP), jnp.float32),
            pltpu.VMEM((K2, LP), jnp.float32),
            pltpu.VMEM((K2, LP), jnp.float32),
            pltpu.VMEM((K2, CH), jnp.float32),
            pltpu.VMEM((_ZROWS, CH), jnp.float32),
            pltpu.VMEM_SHARED((N, CH), jnp.float32),
            pltpu.SemaphoreType.DMA,
            pltpu.SemaphoreType.DMA,
        ],
        compiler_params=_SC_PARAMS,
    )(src3, dst3, hwlo, hwhi, ex, d0, d1)


def kernel(x, q_Y_sample, adj, t, num_steps, W_t1, b_t1, W_t2, b_t2, Wr,
           Wg0, as0, ad0, bg0, Wg1, as1, ad1, bg1, Wg2, as2, ad2, bg2,
           Wf1, bf1, Wf2, bf2):
    f32 = jnp.float32

    tv = t.astype(f32)
    half = NHID // 2
    emb = math.log(10000.0) / (half - 1)
    freqs = jnp.exp(jnp.arange(half, dtype=f32) * -emb)
    a = (tv * 4.0)[:, None] * freqs[None, :]
    pe_abs = jnp.concatenate([jnp.sin(a), jnp.cos(a)], axis=-1)
    pe_abs = jnp.zeros((8, NHID), f32).at[0:1].set(pe_abs)

    inv_freq = 1.0 / (10000.0 ** (jnp.arange(0.0, DIN, 2.0, dtype=f32) / DIN))
    si = tv[:, None] * inv_freq[None, :]
    pe_rel = jnp.concatenate([jnp.sin(si), jnp.cos(si)], axis=-1)
    pe_rel = jnp.zeros((8, HP), f32).at[0:1, :DREL].set(pe_rel)

    w1 = W_t1.astype(f32)
    b1 = jnp.zeros((8, 2 * NHID), f32).at[0].set(b_t1)
    w2 = W_t2.astype(f32)
    b2 = jnp.zeros((8, NHID), f32).at[0].set(b_t2)
    wr_p = jnp.zeros((HP, HW), f32).at[:DREL].set(Wr)

    t_abs, tflat = _prep(pe_abs, w1, b1, w2, b2, pe_rel, wr_p)

    cols_lo = (jnp.arange(HWH) // CH) * NHID + jnp.arange(HWH) % CH
    cols_hi = cols_lo + CH
    rows384 = jnp.arange(HWH) // CH
    mask64 = (rows384[:, None] == jnp.arange(LP)[None, :]).astype(f32)

    tflo = tflat[:, cols_lo]
    tfhi = tflat[:, cols_hi]

    def mk_ws(wg, a_s, a_d):
        wp = jnp.zeros((HP, HW), f32).at[:DIN].set(wg)
        asf = a_s.reshape(HW)
        adf = a_d.reshape(HW)

        def row8(v):
            return jnp.zeros((8, HWH), f32).at[0].set(v)
        return (wp[:, cols_lo], wp[:, cols_hi],
                row8(asf[cols_lo]), row8(asf[cols_hi]),
                row8(adf[cols_lo]), row8(adf[cols_hi]),
                tflo, tfhi, mask64)

    qpad = jnp.zeros((N, NHID), f32).at[:, :NLABEL].set(q_Y_sample)
    hpad0 = jnp.concatenate([x.astype(f32), qpad], axis=1)

    src = adj[0].astype(jnp.int32)
    dst = adj[1].astype(jnp.int32)
    src3 = src.reshape(NWORK, NCHUNK, K)
    dst3 = dst.reshape(NWORK, NCHUNK, K)
    src3b = src.reshape(NS, NCHUNK2, K2)
    dst3b = dst.reshape(NS, NCHUNK2, K2)

    bts = [jnp.zeros((8, NHID), f32).at[0].set(b)[0:1] + t_abs[0:1]
           for b in (bg0, bg1, bg2)]
    bts = [jnp.concatenate([b, jnp.zeros((7, NHID), f32)], axis=0)
           for b in bts]

    layers = [mk_ws(Wg0, as0, ad0), mk_ws(Wg1, as1, ad1), mk_ws(Wg2, as2, ad2)]

    u0 = u1 = None
    for li, ws in enumerate(layers):
        if li == 0:
            hwlo, hwhi, s1p, adp = _layer0(hpad0, *ws)
        else:
            hwlo, hwhi, s1p, adp = _layern(u0, u1, bts[li - 1], qpad, *ws)
        ex, den = _pass1(src3, dst3, s1p, adp)
        up = _pass2(src3b, dst3b, hwlo, hwhi, ex, den[0], den[1])
        u0, u1 = up[0], up[1]

    wf1 = jnp.zeros((HP, 384), f32).at[:FDIM, :2 * FDIM].set(Wf1)
    bf1p = jnp.zeros((8, 384), f32).at[0, :2 * FDIM].set(bf1)
    wf2 = jnp.zeros((384, NHID), f32).at[:2 * FDIM, :2 * NLABEL].set(Wf2)
    bf2p = jnp.zeros((8, NHID), f32).at[0, :2 * NLABEL].set(bf2)

    out = _final(u0, u1, bts[2], qpad, wf1, bf1p, wf2, bf2p)
    return out[:, :2 * NLABEL]


# speedup vs baseline: 49.8767x; 1.0538x over previous
import math

import jax
import jax.numpy as jnp
from jax import lax
from jax.experimental import pallas as pl
from jax.experimental.pallas import tpu as pltpu
from jax.experimental.pallas import tpu_sc as plsc

N = 10000
E = 320000
NFEAT = 128
NLABEL = 5
NHID = 128
NHEAD = 6
DIN = NFEAT + NLABEL
DREL = DIN + 1
FDIM = NHID + NLABEL
HW = NHEAD * NHID
HWH = HW // 2
CH = 64
HP = 256
LP = 16

NC, NS, L = 2, 16, 16
NWORK = NC * NS
EPW = E // NWORK
K = 80
NCHUNK = EPW // K
EPT = E // NS
K2 = 40
NCHUNK2 = EPT // K2
ALN = 624
TAIL = N - NS * ALN

BLK = 2000
GRID = N // BLK


def _prep_body(pe_abs, w1, b1, w2, b2, pe_rel, wr, t_abs_o, tflat_o):
    z = jnp.dot(pe_abs[...], w1[...], preferred_element_type=jnp.float32)
    z = z + b1[...]
    z = jnp.where(z > 0, z, jnp.exp(z) - 1.0)
    ta = jnp.dot(z, w2[...], preferred_element_type=jnp.float32) + b2[...]
    t_abs_o[...] = ta
    tflat_o[...] = jnp.dot(pe_rel[...], wr[...],
                           preferred_element_type=jnp.float32)


def _prep(pe_abs, w1, b1, w2, b2, pe_rel, wr):
    return pl.pallas_call(
        _prep_body,
        out_shape=(jax.ShapeDtypeStruct((8, NHID), jnp.float32),
                   jax.ShapeDtypeStruct((8, HW), jnp.float32)),
    )(pe_abs, w1, b1, w2, b2, pe_rel, wr)


def _matmul_tail(hblk, wlo, whi, aslo, ashi, adlo, adhi, tflo, tfhi, mask64,
                 lo_o, hi_o, s1_o, ad_o):
    lo = jnp.dot(hblk, wlo[...], preferred_element_type=jnp.float32)
    hi = jnp.dot(hblk, whi[...], preferred_element_type=jnp.float32)
    lo_o[...] = lo.astype(jnp.bfloat16)
    hi_o[...] = hi.astype(jnp.bfloat16)
    svlo = aslo[0:1, :] + tflo[0:1, :]
    svhi = ashi[0:1, :] + tfhi[0:1, :]
    s1_o[...] = (jnp.dot(lo * svlo, mask64[...],
                         preferred_element_type=jnp.float32) +
                 jnp.dot(hi * svhi, mask64[...],
                         preferred_element_type=jnp.float32))
    ad_o[...] = (jnp.dot(lo * adlo[0:1, :], mask64[...],
                         preferred_element_type=jnp.float32) +
                 jnp.dot(hi * adhi[0:1, :], mask64[...],
                         preferred_element_type=jnp.float32))


def _layer0_body(hpad, wlo, whi, aslo, ashi, adlo, adhi, tflo, tfhi, mask64,
                 lo_o, hi_o, s1_o, ad_o):
    _matmul_tail(hpad[...], wlo, whi, aslo, ashi, adlo, adhi, tflo, tfhi,
                 mask64, lo_o, hi_o, s1_o, ad_o)


def _layern_body(u0, u1, bt, qpad, wlo, whi, aslo, ashi, adlo, adhi,
                 tflo, tfhi, mask64, lo_o, hi_o, s1_o, ad_o):
    g = jnp.concatenate([u0[...], u1[...]], axis=1) * (1.0 / NHEAD)
    g = g + bt[0:1, :]
    e = jnp.where(g > 0, g, jnp.exp(g) - 1.0)
    hblk = jnp.concatenate([e, qpad[...]], axis=1)
    _matmul_tail(hblk, wlo, whi, aslo, ashi, adlo, adhi, tflo, tfhi,
                 mask64, lo_o, hi_o, s1_o, ad_o)


def _row_spec(w):
    return pl.BlockSpec((BLK, w), lambda i: (i, 0))


def _full_spec(r, c):
    return pl.BlockSpec((r, c), lambda i: (0, 0))


_LAYER_OUT = (jax.ShapeDtypeStruct((N, HWH), jnp.bfloat16),
              jax.ShapeDtypeStruct((N, HWH), jnp.bfloat16),
              jax.ShapeDtypeStruct((N, LP), jnp.float32),
              jax.ShapeDtypeStruct((N, LP), jnp.float32))
_LAYER_OUT_SPECS = (_row_spec(HWH), _row_spec(HWH),
                    _row_spec(LP), _row_spec(LP))
_WSPECS = [_full_spec(HP, HWH), _full_spec(HP, HWH), _full_spec(8, HWH),
           _full_spec(8, HWH), _full_spec(8, HWH), _full_spec(8, HWH),
           _full_spec(8, HWH), _full_spec(8, HWH), _full_spec(HWH, LP)]


def _layer0(hpad, *ws):
    return pl.pallas_call(
        _layer0_body,
        grid=(GRID,),
        in_specs=[_row_spec(HP)] + _WSPECS,
        out_specs=_LAYER_OUT_SPECS,
        out_shape=_LAYER_OUT,
    )(hpad, *ws)


def _layern(u0, u1, bt, qpad, *ws):
    return pl.pallas_call(
        _layern_body,
        grid=(GRID,),
        in_specs=[_row_spec(CH), _row_spec(CH), _full_spec(8, NHID),
                  _row_spec(NHID)] + _WSPECS,
        out_specs=_LAYER_OUT_SPECS,
        out_shape=_LAYER_OUT,
    )(u0, u1, bt, qpad, *ws)


def _final_body(u0, u1, bt, qpad, wf1, bf1, wf2, bf2, out_o):
    g = jnp.concatenate([u0[...], u1[...]], axis=1) * (1.0 / NHEAD)
    g = g + bt[0:1, :]
    e = jnp.where(g > 0, g, jnp.exp(g) - 1.0)
    hblk = jnp.concatenate([e, qpad[...]], axis=1)
    z = jnp.dot(hblk, wf1[...], preferred_element_type=jnp.float32)
    z = z + bf1[0:1, :]
    z = jnp.where(z > 0, z, jnp.exp(z) - 1.0)
    out_o[...] = jnp.dot(z, wf2[...],
                         preferred_element_type=jnp.float32) + bf2[0:1, :]


def _final(u0, u1, bt, qpad, wf1, bf1, wf2, bf2):
    return pl.pallas_call(
        _final_body,
        grid=(GRID,),
        in_specs=[_row_spec(CH), _row_spec(CH), _full_spec(8, NHID),
                  _row_spec(NHID), _full_spec(HP, 384), _full_spec(8, 384),
                  _full_spec(384, NHID), _full_spec(8, NHID)],
        out_specs=_row_spec(NHID),
        out_shape=jax.ShapeDtypeStruct((N, NHID), jnp.float32),
    )(u0, u1, bt, qpad, wf1, bf1, wf2, bf2)


_MESH = plsc.VectorSubcoreMesh(core_axis_name="c", subcore_axis_name="s",
                               num_cores=NC, num_subcores=NS)
_SC_PARAMS = pltpu.CompilerParams(use_tc_tiling_on_sc=False,
                                  needs_layout_passes=False)


def _pass1_body(src_h, dst_h, s1_h, ad_h, ex_h, den_h,
                src_pf, dst_pf, s1_a, s1_b, ad_a, ad_b, ex_v, zb, den_sp,
                sem_a, sem_b):
    cid = lax.axis_index("c")
    sid = lax.axis_index("s")
    wid = sid * NC + cid
    s1_v = (s1_a, s1_b)
    ad_v = (ad_a, ad_b)
    sem = (sem_a, sem_b)

    def zrow(i, _):
        zb[i, :] = jnp.zeros((L,), jnp.float32)
        return 0
    lax.fori_loop(0, ALN, zrow, 0)
    pltpu.sync_copy(zb, den_sp.at[pl.ds(sid * ALN, ALN)])

    @pl.when(sid == 0)
    def _():
        pltpu.sync_copy(zb.at[pl.ds(0, TAIL)],
                        den_sp.at[pl.ds(NS * ALN, TAIL)])

    pltpu.sync_copy(src_h.at[wid], src_pf)
    pltpu.sync_copy(dst_h.at[wid], dst_pf)
    plsc.subcore_barrier()

    def fire(c, b):
        pltpu.async_copy(s1_h.at[src_pf.at[c]], s1_v[b], sem[b])
        pltpu.async_copy(ad_h.at[dst_pf.at[c]], ad_v[b], sem[b])

    def drain(c, b):
        pltpu.make_async_copy(s1_h.at[src_pf.at[c]], s1_v[b], sem[b]).wait()
        pltpu.make_async_copy(ad_h.at[dst_pf.at[c]], ad_v[b], sem[b]).wait()

    def work(c, b):
        def edge(i, _):
            v = s1_v[b][i, :] + ad_v[b][i, :]
            v = jnp.where(v >= 0, v, v * 0.2)
            ex_v[i, :] = jnp.exp(v)
            return 0
        lax.fori_loop(0, K, edge, 0)
        pltpu.sync_copy(ex_v, den_sp.at[dst_pf.at[c]], add=True)
        pltpu.sync_copy(ex_v, ex_h.at[pl.ds(wid * EPW + c * K, K)])

    fire(0, 0)

    def pair(g, _):
        c0 = 2 * g
        fire(c0 + 1, 1)
        drain(c0, 0)
        work(c0, 0)
        fire(c0 + 2, 0)
        drain(c0 + 1, 1)
        work(c0 + 1, 1)
        return 0
    lax.fori_loop(0, (NCHUNK - 1) // 2, pair, 0)
    drain(NCHUNK - 1, 0)
    work(NCHUNK - 1, 0)

    plsc.subcore_barrier()
    pltpu.sync_copy(den_sp.at[pl.ds(sid * ALN, ALN)],
                    den_h.at[cid, pl.ds(sid * ALN, ALN)])

    @pl.when(sid == 0)
    def _():
        pltpu.sync_copy(den_sp.at[pl.ds(NS * ALN, TAIL)],
                        den_h.at[cid, pl.ds(NS * ALN, TAIL)])


def _pass1(src3, dst3, s1p, adp):
    return pl.kernel(
        _pass1_body,
        out_type=(jax.ShapeDtypeStruct((E, LP), jnp.float32),
                  jax.ShapeDtypeStruct((NC, N, LP), jnp.float32)),
        mesh=_MESH,
        scratch_types=[
            pltpu.VMEM((NCHUNK, K), jnp.int32),
            pltpu.VMEM((NCHUNK, K), jnp.int32),
            pltpu.VMEM((K, LP), jnp.float32),
            pltpu.VMEM((K, LP), jnp.float32),
            pltpu.VMEM((K, LP), jnp.float32),
            pltpu.VMEM((K, LP), jnp.float32),
            pltpu.VMEM((K, LP), jnp.float32),
            pltpu.VMEM((ALN, LP), jnp.float32),
            pltpu.VMEM_SHARED((N, LP), jnp.float32),
            pltpu.SemaphoreType.DMA,
            pltpu.SemaphoreType.DMA,
        ],
        compiler_params=_SC_PARAMS,
    )(src3, dst3, s1p, adp)


_ZROWS = 48


def _pass2_body(src_h, dst_h, lo_h, hi_h, ex_h, d0_h, d1_h, up_h,
                src_pf, dst_pf, rows_a, rows_b, ex_a, ex_b,
                d0_a, d0_b, d1_a, d1_b, msg_v, zb, out_sp, sem_a, sem_b):
    cid = lax.axis_index("c")
    sid = lax.axis_index("s")
    lane = lax.broadcasted_iota(jnp.int32, (L,), 0)
    rows_v = (rows_a, rows_b)
    ex_v = (ex_a, ex_b)
    d0_v = (d0_a, d0_b)
    d1_v = (d1_a, d1_b)
    sem = (sem_a, sem_b)

    def zrow(i, _):
        for j in range(CH // L):
            zb[i, pl.ds(j * L, L)] = jnp.zeros((L,), jnp.float32)
        return 0
    lax.fori_loop(0, _ZROWS, zrow, 0)
    for kk in range(ALN // _ZROWS):
        pltpu.sync_copy(zb, out_sp.at[pl.ds(sid * ALN + kk * _ZROWS, _ZROWS)])

    @pl.when(sid == 0)
    def _():
        pltpu.sync_copy(zb.at[pl.ds(0, TAIL)],
                        out_sp.at[pl.ds(NS * ALN, TAIL)])

    pltpu.sync_copy(src_h.at[sid], src_pf)
    pltpu.sync_copy(dst_h.at[sid], dst_pf)
    plsc.subcore_barrier()

    def sweep(tab_h):
        def fire(c, b):
            pltpu.async_copy(tab_h.at[src_pf.at[c]], rows_v[b], sem[b])
            pltpu.async_copy(ex_h.at[pl.ds(sid * EPT + c * K2, K2)],
                             ex_v[b], sem[b])
            pltpu.async_copy(d0_h.at[dst_pf.at[c]], d0_v[b], sem[b])
            pltpu.async_copy(d1_h.at[dst_pf.at[c]], d1_v[b], sem[b])

        def drain(c, b):
            pltpu.make_async_copy(tab_h.at[src_pf.at[c]], rows_v[b],
                                  sem[b]).wait()
            pltpu.make_async_copy(ex_h.at[pl.ds(sid * EPT + c * K2, K2)],
                                  ex_v[b], sem[b]).wait()
            pltpu.make_async_copy(d0_h.at[dst_pf.at[c]], d0_v[b],
                                  sem[b]).wait()
            pltpu.make_async_copy(d1_h.at[dst_pf.at[c]], d1_v[b],
                                  sem[b]).wait()

        def work(c, b):
            def edge(i, _):
                w = ex_v[b][i, :] / (d0_v[b][i, :] + d1_v[b][i, :] + 1e-16)
                acc = [None] * (CH // L)
                for h in range(NHEAD):
                    wh = lax.gather(
                        w, (lane * 0 + h)[:, None],
                        lax.GatherDimensionNumbers(
                            offset_dims=(), collapsed_slice_dims=(0,),
                            start_index_map=(0,)),
                        (1,),
                        mode=lax.GatherScatterMode.PROMISE_IN_BOUNDS)
                    for g2 in range(CH // (2 * L)):
                        ab = rows_v[b][i, pl.ds(h * CH + g2 * 2 * L, 2 * L)]
                        pa, pb = plsc.unpack(
                            ab, format=plsc.PackFormat.INTERLEAVED,
                            preferred_element_type=jnp.float32)
                        j = 2 * g2
                        if h == 0:
                            acc[j] = wh * pa
                            acc[j + 1] = wh * pb
                        else:
                            acc[j] = acc[j] + wh * pa
                            acc[j + 1] = acc[j + 1] + wh * pb
                for j in range(CH // L):
                    msg_v[i, pl.ds(j * L, L)] = acc[j]
                return 0
            lax.fori_loop(0, K2, edge, 0)
            pltpu.sync_copy(msg_v, out_sp.at[dst_pf.at[c]], add=True)

        fire(0, 0)

        def pair(g, _):
            c0 = 2 * g
            fire(c0 + 1, 1)
            drain(c0, 0)
            work(c0, 0)
            fire(c0 + 2, 0)
            drain(c0 + 1, 1)
            work(c0 + 1, 1)
            return 0
        lax.fori_loop(0, (NCHUNK2 - 2) // 2, pair, 0)
        c0 = NCHUNK2 - 2
        fire(c0 + 1, 1)
        drain(c0, 0)
        work(c0, 0)
        drain(c0 + 1, 1)
        work(c0 + 1, 1)

    @pl.when(cid == 0)
    def _():
        sweep(lo_h)

    @pl.when(cid == 1)
    def _():
        sweep(hi_h)

    plsc.subcore_barrier()
    pltpu.sync_copy(out_sp.at[pl.ds(sid * ALN, ALN)],
                    up_h.at[cid, pl.ds(sid * ALN, ALN)])

    @pl.when(sid == 0)
    def _():
        pltpu.sync_copy(out_sp.at[pl.ds(NS * ALN, TAIL)],
                        up_h.at[cid, pl.ds(NS * ALN, TAIL)])


def _pass2(src3, dst3, hwlo, hwhi, ex, d0, d1):
    return pl.kernel(
        _pass2_body,
        out_type=jax.ShapeDtypeStruct((NC, N, CH), jnp.float32),
        mesh=_MESH,
        scratch_types=[
            pltpu.VMEM((NCHUNK2, K2), jnp.int32),
            pltpu.VMEM((NCHUNK2, K2), jnp.int32),
            pltpu.VMEM((K2, HWH), jnp.bfloat16),
            pltpu.VMEM((K2, HWH), jnp.bfloat16),
            pltpu.VMEM((K2, LP), jnp.float32),
            pltpu.VMEM((K2, LP), jnp.float32),
            pltpu.VMEM((K2, LP), jnp.float32),
            pltpu.VMEM((K2, LP), jnp.float32),
            pltpu.VMEM((K2, LP), jnp.float32),
            pltpu.VMEM((K2, LP), jnp.float32),
            pltpu.VMEM((K2, CH), jnp.float32),
            pltpu.VMEM((_ZROWS, CH), jnp.float32),
            pltpu.VMEM_SHARED((N, CH), jnp.float32),
            pltpu.SemaphoreType.DMA,
            pltpu.SemaphoreType.DMA,
        ],
        compiler_params=_SC_PARAMS,
    )(src3, dst3, hwlo, hwhi, ex, d0, d1)


def kernel(x, q_Y_sample, adj, t, num_steps, W_t1, b_t1, W_t2, b_t2, Wr,
           Wg0, as0, ad0, bg0, Wg1, as1, ad1, bg1, Wg2, as2, ad2, bg2,
           Wf1, bf1, Wf2, bf2):
    f32 = jnp.float32

    tv = t.astype(f32)
    half = NHID // 2
    emb = math.log(10000.0) / (half - 1)
    freqs = jnp.exp(jnp.arange(half, dtype=f32) * -emb)
    a = (tv * 4.0)[:, None] * freqs[None, :]
    pe_abs = jnp.concatenate([jnp.sin(a), jnp.cos(a)], axis=-1)
    pe_abs = jnp.zeros((8, NHID), f32).at[0:1].set(pe_abs)

    inv_freq = 1.0 / (10000.0 ** (jnp.arange(0.0, DIN, 2.0, dtype=f32) / DIN))
    si = tv[:, None] * inv_freq[None, :]
    pe_rel = jnp.concatenate([jnp.sin(si), jnp.cos(si)], axis=-1)
    pe_rel = jnp.zeros((8, HP), f32).at[0:1, :DREL].set(pe_rel)

    w1 = W_t1.astype(f32)
    b1 = jnp.zeros((8, 2 * NHID), f32).at[0].set(b_t1)
    w2 = W_t2.astype(f32)
    b2 = jnp.zeros((8, NHID), f32).at[0].set(b_t2)
    wr_p = jnp.zeros((HP, HW), f32).at[:DREL].set(Wr)

    t_abs, tflat = _prep(pe_abs, w1, b1, w2, b2, pe_rel, wr_p)

    pos = jnp.arange(HWH)
    chan = (pos // 32) * 32 + (pos % 2) * L + (pos % 32) // 2
    base_lo = (chan // CH) * NHID + chan % CH
    cols_lo = base_lo
    cols_hi = base_lo + CH
    rows384 = jnp.arange(HWH) // CH
    mask64 = (rows384[:, None] == jnp.arange(LP)[None, :]).astype(f32)

    tflo = tflat[:, cols_lo]
    tfhi = tflat[:, cols_hi]

    def mk_ws(wg, a_s, a_d):
        wp = jnp.zeros((HP, HW), f32).at[:DIN].set(wg)
        asf = a_s.reshape(HW)
        adf = a_d.reshape(HW)

        def row8(v):
            return jnp.zeros((8, HWH), f32).at[0].set(v)
        return (wp[:, cols_lo], wp[:, cols_hi],
                row8(asf[cols_lo]), row8(asf[cols_hi]),
                row8(adf[cols_lo]), row8(adf[cols_hi]),
                tflo, tfhi, mask64)

    qpad = jnp.zeros((N, NHID), f32).at[:, :NLABEL].set(q_Y_sample)
    hpad0 = jnp.concatenate([x.astype(f32), qpad], axis=1)

    src = adj[0].astype(jnp.int32)
    dst = adj[1].astype(jnp.int32)
    src3 = src.reshape(NWORK, NCHUNK, K)
    dst3 = dst.reshape(NWORK, NCHUNK, K)
    src3b = src.reshape(NS, NCHUNK2, K2)
    dst3b = dst.reshape(NS, NCHUNK2, K2)

    bts = [jnp.zeros((8, NHID), f32).at[0].set(b)[0:1] + t_abs[0:1]
           for b in (bg0, bg1, bg2)]
    bts = [jnp.concatenate([b, jnp.zeros((7, NHID), f32)], axis=0)
           for b in bts]

    layers = [mk_ws(Wg0, as0, ad0), mk_ws(Wg1, as1, ad1), mk_ws(Wg2, as2, ad2)]

    u0 = u1 = None
    for li, ws in enumerate(layers):
        if li == 0:
            hwlo, hwhi, s1p, adp = _layer0(hpad0, *ws)
        else:
            hwlo, hwhi, s1p, adp = _layern(u0, u1, bts[li - 1], qpad, *ws)
        ex, den = _pass1(src3, dst3, s1p, adp)
        up = _pass2(src3b, dst3b, hwlo, hwhi, ex, den[0], den[1])
        u0, u1 = up[0], up[1]

    wf1 = jnp.zeros((HP, 384), f32).at[:FDIM, :2 * FDIM].set(Wf1)
    bf1p = jnp.zeros((8, 384), f32).at[0, :2 * FDIM].set(bf1)
    wf2 = jnp.zeros((384, NHID), f32).at[:2 * FDIM, :2 * NLABEL].set(Wf2)
    bf2p = jnp.zeros((8, NHID), f32).at[0, :2 * NLABEL].set(bf2)

    out = _final(u0, u1, bts[2], qpad, wf1, bf1p, wf2, bf2p)
    return out[:, :2 * NLABEL]


# speedup vs baseline: 51.1832x; 1.0262x over previous
import math

import jax
import jax.numpy as jnp
from jax import lax
from jax.experimental import pallas as pl
from jax.experimental.pallas import tpu as pltpu
from jax.experimental.pallas import tpu_sc as plsc

N = 10000
E = 320000
NFEAT = 128
NLABEL = 5
NHID = 128
NHEAD = 6
DIN = NFEAT + NLABEL
DREL = DIN + 1
FDIM = NHID + NLABEL
HW = NHEAD * NHID
HWH = HW // 2
CH = 64
HP = 256
LP = 16

NC, NS, L = 2, 16, 16
NWORK = NC * NS
EPW = E // NWORK
K = 80
NCHUNK = EPW // K
EPT = E // NS
K2 = 80
NCHUNK2 = EPT // K2
ALN = 624
TAIL = N - NS * ALN

BLK = 2000
GRID = N // BLK


def _prep_body(pe_abs, w1, b1, w2, b2, pe_rel, wr, t_abs_o, tflat_o):
    z = jnp.dot(pe_abs[...], w1[...], preferred_element_type=jnp.float32)
    z = z + b1[...]
    z = jnp.where(z > 0, z, jnp.exp(z) - 1.0)
    ta = jnp.dot(z, w2[...], preferred_element_type=jnp.float32) + b2[...]
    t_abs_o[...] = ta
    tflat_o[...] = jnp.dot(pe_rel[...], wr[...],
                           preferred_element_type=jnp.float32)


def _prep(pe_abs, w1, b1, w2, b2, pe_rel, wr):
    return pl.pallas_call(
        _prep_body,
        out_shape=(jax.ShapeDtypeStruct((8, NHID), jnp.float32),
                   jax.ShapeDtypeStruct((8, HW), jnp.float32)),
    )(pe_abs, w1, b1, w2, b2, pe_rel, wr)


def _matmul_tail(hblk, wlo, whi, aslo, ashi, adlo, adhi, tflo, tfhi, mask64,
                 lo_o, hi_o, s1_o, ad_o):
    lo = jnp.dot(hblk, wlo[...], preferred_element_type=jnp.float32)
    hi = jnp.dot(hblk, whi[...], preferred_element_type=jnp.float32)
    lo_o[...] = lo.astype(jnp.bfloat16)
    hi_o[...] = hi.astype(jnp.bfloat16)
    svlo = aslo[0:1, :] + tflo[0:1, :]
    svhi = ashi[0:1, :] + tfhi[0:1, :]
    s1_o[...] = (jnp.dot(lo * svlo, mask64[...],
                         preferred_element_type=jnp.float32) +
                 jnp.dot(hi * svhi, mask64[...],
                         preferred_element_type=jnp.float32))
    ad_o[...] = (jnp.dot(lo * adlo[0:1, :], mask64[...],
                         preferred_element_type=jnp.float32) +
                 jnp.dot(hi * adhi[0:1, :], mask64[...],
                         preferred_element_type=jnp.float32))


def _layer0_body(hpad, wlo, whi, aslo, ashi, adlo, adhi, tflo, tfhi, mask64,
                 lo_o, hi_o, s1_o, ad_o):
    _matmul_tail(hpad[...], wlo, whi, aslo, ashi, adlo, adhi, tflo, tfhi,
                 mask64, lo_o, hi_o, s1_o, ad_o)


def _layern_body(u0, u1, bt, qpad, wlo, whi, aslo, ashi, adlo, adhi,
                 tflo, tfhi, mask64, lo_o, hi_o, s1_o, ad_o):
    g = jnp.concatenate([u0[...], u1[...]], axis=1) * (1.0 / NHEAD)
    g = g + bt[0:1, :]
    e = jnp.where(g > 0, g, jnp.exp(g) - 1.0)
    hblk = jnp.concatenate([e, qpad[...]], axis=1)
    _matmul_tail(hblk, wlo, whi, aslo, ashi, adlo, adhi, tflo, tfhi,
                 mask64, lo_o, hi_o, s1_o, ad_o)


def _row_spec(w):
    return pl.BlockSpec((BLK, w), lambda i: (i, 0))


def _full_spec(r, c):
    return pl.BlockSpec((r, c), lambda i: (0, 0))


_LAYER_OUT = (jax.ShapeDtypeStruct((N, HWH), jnp.bfloat16),
              jax.ShapeDtypeStruct((N, HWH), jnp.bfloat16),
              jax.ShapeDtypeStruct((N, LP), jnp.float32),
              jax.ShapeDtypeStruct((N, LP), jnp.float32))
_LAYER_OUT_SPECS = (_row_spec(HWH), _row_spec(HWH),
                    _row_spec(LP), _row_spec(LP))
_WSPECS = [_full_spec(HP, HWH), _full_spec(HP, HWH), _full_spec(8, HWH),
           _full_spec(8, HWH), _full_spec(8, HWH), _full_spec(8, HWH),
           _full_spec(8, HWH), _full_spec(8, HWH), _full_spec(HWH, LP)]


def _layer0(hpad, *ws):
    return pl.pallas_call(
        _layer0_body,
        grid=(GRID,),
        in_specs=[_row_spec(HP)] + _WSPECS,
        out_specs=_LAYER_OUT_SPECS,
        out_shape=_LAYER_OUT,
    )(hpad, *ws)


def _layern(u0, u1, bt, qpad, *ws):
    return pl.pallas_call(
        _layern_body,
        grid=(GRID,),
        in_specs=[_row_spec(CH), _row_spec(CH), _full_spec(8, NHID),
                  _row_spec(NHID)] + _WSPECS,
        out_specs=_LAYER_OUT_SPECS,
        out_shape=_LAYER_OUT,
    )(u0, u1, bt, qpad, *ws)


def _final_body(u0, u1, bt, qpad, wf1, bf1, wf2, bf2, out_o):
    g = jnp.concatenate([u0[...], u1[...]], axis=1) * (1.0 / NHEAD)
    g = g + bt[0:1, :]
    e = jnp.where(g > 0, g, jnp.exp(g) - 1.0)
    hblk = jnp.concatenate([e, qpad[...]], axis=1)
    z = jnp.dot(hblk, wf1[...], preferred_element_type=jnp.float32)
    z = z + bf1[0:1, :]
    z = jnp.where(z > 0, z, jnp.exp(z) - 1.0)
    out_o[...] = jnp.dot(z, wf2[...],
                         preferred_element_type=jnp.float32) + bf2[0:1, :]


def _final(u0, u1, bt, qpad, wf1, bf1, wf2, bf2):
    return pl.pallas_call(
        _final_body,
        grid=(GRID,),
        in_specs=[_row_spec(CH), _row_spec(CH), _full_spec(8, NHID),
                  _row_spec(NHID), _full_spec(HP, 384), _full_spec(8, 384),
                  _full_spec(384, NHID), _full_spec(8, NHID)],
        out_specs=_row_spec(NHID),
        out_shape=jax.ShapeDtypeStruct((N, NHID), jnp.float32),
    )(u0, u1, bt, qpad, wf1, bf1, wf2, bf2)


_MESH = plsc.VectorSubcoreMesh(core_axis_name="c", subcore_axis_name="s",
                               num_cores=NC, num_subcores=NS)
_SC_PARAMS = pltpu.CompilerParams(use_tc_tiling_on_sc=False,
                                  needs_layout_passes=False)


def _pass1_body(src_h, dst_h, s1_h, ad_h, ex_h, den_h,
                src_pf, dst_pf, s1_a, s1_b, ad_a, ad_b, ex_v, zb, den_sp,
                sem_a, sem_b):
    cid = lax.axis_index("c")
    sid = lax.axis_index("s")
    wid = sid * NC + cid
    s1_v = (s1_a, s1_b)
    ad_v = (ad_a, ad_b)
    sem = (sem_a, sem_b)

    def zrow(i, _):
        zb[i, :] = jnp.zeros((L,), jnp.float32)
        return 0
    lax.fori_loop(0, ALN, zrow, 0)
    pltpu.sync_copy(zb, den_sp.at[pl.ds(sid * ALN, ALN)])

    @pl.when(sid == 0)
    def _():
        pltpu.sync_copy(zb.at[pl.ds(0, TAIL)],
                        den_sp.at[pl.ds(NS * ALN, TAIL)])

    pltpu.sync_copy(src_h.at[wid], src_pf)
    pltpu.sync_copy(dst_h.at[wid], dst_pf)
    plsc.subcore_barrier()

    def fire(c, b):
        pltpu.async_copy(s1_h.at[src_pf.at[c]], s1_v[b], sem[b])
        pltpu.async_copy(ad_h.at[dst_pf.at[c]], ad_v[b], sem[b])

    def drain(c, b):
        pltpu.make_async_copy(s1_h.at[src_pf.at[c]], s1_v[b], sem[b]).wait()
        pltpu.make_async_copy(ad_h.at[dst_pf.at[c]], ad_v[b], sem[b]).wait()

    def work(c, b):
        def edge(i, _):
            v = s1_v[b][i, :] + ad_v[b][i, :]
            v = jnp.where(v >= 0, v, v * 0.2)
            ex_v[i, :] = jnp.exp(v)
            return 0
        lax.fori_loop(0, K, edge, 0)
        pltpu.sync_copy(ex_v, den_sp.at[dst_pf.at[c]], add=True)
        pltpu.sync_copy(ex_v, ex_h.at[pl.ds(wid * EPW + c * K, K)])

    fire(0, 0)

    def pair(g, _):
        c0 = 2 * g
        fire(c0 + 1, 1)
        drain(c0, 0)
        work(c0, 0)
        fire(c0 + 2, 0)
        drain(c0 + 1, 1)
        work(c0 + 1, 1)
        return 0
    lax.fori_loop(0, (NCHUNK - 1) // 2, pair, 0)
    drain(NCHUNK - 1, 0)
    work(NCHUNK - 1, 0)

    plsc.subcore_barrier()
    pltpu.sync_copy(den_sp.at[pl.ds(sid * ALN, ALN)],
                    den_h.at[cid, pl.ds(sid * ALN, ALN)])

    @pl.when(sid == 0)
    def _():
        pltpu.sync_copy(den_sp.at[pl.ds(NS * ALN, TAIL)],
                        den_h.at[cid, pl.ds(NS * ALN, TAIL)])


def _pass1(src3, dst3, s1p, adp):
    return pl.kernel(
        _pass1_body,
        out_type=(jax.ShapeDtypeStruct((E, LP), jnp.float32),
                  jax.ShapeDtypeStruct((NC, N, LP), jnp.float32)),
        mesh=_MESH,
        scratch_types=[
            pltpu.VMEM((NCHUNK, K), jnp.int32),
            pltpu.VMEM((NCHUNK, K), jnp.int32),
            pltpu.VMEM((K, LP), jnp.float32),
            pltpu.VMEM((K, LP), jnp.float32),
            pltpu.VMEM((K, LP), jnp.float32),
            pltpu.VMEM((K, LP), jnp.float32),
            pltpu.VMEM((K, LP), jnp.float32),
            pltpu.VMEM((ALN, LP), jnp.float32),
            pltpu.VMEM_SHARED((N, LP), jnp.float32),
            pltpu.SemaphoreType.DMA,
            pltpu.SemaphoreType.DMA,
        ],
        compiler_params=_SC_PARAMS,
    )(src3, dst3, s1p, adp)


_ZROWS = 48


def _pass2_body(src_h, dst_h, lo_h, hi_h, ex_h, d0_h, d1_h, up_h,
                src_pf, dst_pf, rows_a, rows_b, ex_a, ex_b,
                d0_a, d0_b, d1_a, d1_b, msg_v, zb, out_sp, sem_a, sem_b):
    cid = lax.axis_index("c")
    sid = lax.axis_index("s")
    lane = lax.broadcasted_iota(jnp.int32, (L,), 0)
    rows_v = (rows_a, rows_b)
    ex_v = (ex_a, ex_b)
    d0_v = (d0_a, d0_b)
    d1_v = (d1_a, d1_b)
    sem = (sem_a, sem_b)

    def zrow(i, _):
        for j in range(CH // L):
            zb[i, pl.ds(j * L, L)] = jnp.zeros((L,), jnp.float32)
        return 0
    lax.fori_loop(0, _ZROWS, zrow, 0)
    for kk in range(ALN // _ZROWS):
        pltpu.sync_copy(zb, out_sp.at[pl.ds(sid * ALN + kk * _ZROWS, _ZROWS)])

    @pl.when(sid == 0)
    def _():
        pltpu.sync_copy(zb.at[pl.ds(0, TAIL)],
                        out_sp.at[pl.ds(NS * ALN, TAIL)])

    pltpu.sync_copy(src_h.at[sid], src_pf)
    pltpu.sync_copy(dst_h.at[sid], dst_pf)
    plsc.subcore_barrier()

    def sweep(tab_h):
        def fire(c, b):
            pltpu.async_copy(tab_h.at[src_pf.at[c]], rows_v[b], sem[b])
            pltpu.async_copy(ex_h.at[pl.ds(sid * EPT + c * K2, K2)],
                             ex_v[b], sem[b])
            pltpu.async_copy(d0_h.at[dst_pf.at[c]], d0_v[b], sem[b])
            pltpu.async_copy(d1_h.at[dst_pf.at[c]], d1_v[b], sem[b])

        def drain(c, b):
            pltpu.make_async_copy(tab_h.at[src_pf.at[c]], rows_v[b],
                                  sem[b]).wait()
            pltpu.make_async_copy(ex_h.at[pl.ds(sid * EPT + c * K2, K2)],
                                  ex_v[b], sem[b]).wait()
            pltpu.make_async_copy(d0_h.at[dst_pf.at[c]], d0_v[b],
                                  sem[b]).wait()
            pltpu.make_async_copy(d1_h.at[dst_pf.at[c]], d1_v[b],
                                  sem[b]).wait()

        def work(c, b):
            def edge(i, _):
                w = ex_v[b][i, :] / (d0_v[b][i, :] + d1_v[b][i, :] + 1e-16)
                acc = [None] * (CH // L)
                for h in range(NHEAD):
                    wh = lax.gather(
                        w, (lane * 0 + h)[:, None],
                        lax.GatherDimensionNumbers(
                            offset_dims=(), collapsed_slice_dims=(0,),
                            start_index_map=(0,)),
                        (1,),
                        mode=lax.GatherScatterMode.PROMISE_IN_BOUNDS)
                    for g2 in range(CH // (2 * L)):
                        ab = rows_v[b][i, pl.ds(h * CH + g2 * 2 * L, 2 * L)]
                        pa, pb = plsc.unpack(
                            ab, format=plsc.PackFormat.INTERLEAVED,
                            preferred_element_type=jnp.float32)
                        j = 2 * g2
                        if h == 0:
                            acc[j] = wh * pa
                            acc[j + 1] = wh * pb
                        else:
                            acc[j] = acc[j] + wh * pa
                            acc[j + 1] = acc[j + 1] + wh * pb
                for j in range(CH // L):
                    msg_v[i, pl.ds(j * L, L)] = acc[j]
                return 0
            lax.fori_loop(0, K2, edge, 0)
            pltpu.sync_copy(msg_v, out_sp.at[dst_pf.at[c]], add=True)

        fire(0, 0)

        def pair(g, _):
            c0 = 2 * g
            fire(c0 + 1, 1)
            drain(c0, 0)
            work(c0, 0)
            fire(c0 + 2, 0)
            drain(c0 + 1, 1)
            work(c0 + 1, 1)
            return 0
        lax.fori_loop(0, (NCHUNK2 - 2) // 2, pair, 0)
        c0 = NCHUNK2 - 2
        fire(c0 + 1, 1)
        drain(c0, 0)
        work(c0, 0)
        drain(c0 + 1, 1)
        work(c0 + 1, 1)

    @pl.when(cid == 0)
    def _():
        sweep(lo_h)

    @pl.when(cid == 1)
    def _():
        sweep(hi_h)

    plsc.subcore_barrier()
    pltpu.sync_copy(out_sp.at[pl.ds(sid * ALN, ALN)],
                    up_h.at[cid, pl.ds(sid * ALN, ALN)])

    @pl.when(sid == 0)
    def _():
        pltpu.sync_copy(out_sp.at[pl.ds(NS * ALN, TAIL)],
                        up_h.at[cid, pl.ds(NS * ALN, TAIL)])


def _pass2(src3, dst3, hwlo, hwhi, ex, d0, d1):
    return pl.kernel(
        _pass2_body,
        out_type=jax.ShapeDtypeStruct((NC, N, CH), jnp.float32),
        mesh=_MESH,
        scratch_types=[
            pltpu.VMEM((NCHUNK2, K2), jnp.int32),
            pltpu.VMEM((NCHUNK2, K2), jnp.int32),
            pltpu.VMEM((K2, HWH), jnp.bfloat16),
            pltpu.VMEM((K2, HWH), jnp.bfloat16),
            pltpu.VMEM((K2, LP), jnp.float32),
            pltpu.VMEM((K2, LP), jnp.float32),
            pltpu.VMEM((K2, LP), jnp.float32),
            pltpu.VMEM((K2, LP), jnp.float32),
            pltpu.VMEM((K2, LP), jnp.float32),
            pltpu.VMEM((K2, LP), jnp.float32),
            pltpu.VMEM((K2, CH), jnp.float32),
            pltpu.VMEM((_ZROWS, CH), jnp.float32),
            pltpu.VMEM_SHARED((N, CH), jnp.float32),
            pltpu.SemaphoreType.DMA,
            pltpu.SemaphoreType.DMA,
        ],
        compiler_params=_SC_PARAMS,
    )(src3, dst3, hwlo, hwhi, ex, d0, d1)


def kernel(x, q_Y_sample, adj, t, num_steps, W_t1, b_t1, W_t2, b_t2, Wr,
           Wg0, as0, ad0, bg0, Wg1, as1, ad1, bg1, Wg2, as2, ad2, bg2,
           Wf1, bf1, Wf2, bf2):
    f32 = jnp.float32

    tv = t.astype(f32)
    half = NHID // 2
    emb = math.log(10000.0) / (half - 1)
    freqs = jnp.exp(jnp.arange(half, dtype=f32) * -emb)
    a = (tv * 4.0)[:, None] * freqs[None, :]
    pe_abs = jnp.concatenate([jnp.sin(a), jnp.cos(a)], axis=-1)
    pe_abs = jnp.zeros((8, NHID), f32).at[0:1].set(pe_abs)

    inv_freq = 1.0 / (10000.0 ** (jnp.arange(0.0, DIN, 2.0, dtype=f32) / DIN))
    si = tv[:, None] * inv_freq[None, :]
    pe_rel = jnp.concatenate([jnp.sin(si), jnp.cos(si)], axis=-1)
    pe_rel = jnp.zeros((8, HP), f32).at[0:1, :DREL].set(pe_rel)

    w1 = W_t1.astype(f32)
    b1 = jnp.zeros((8, 2 * NHID), f32).at[0].set(b_t1)
    w2 = W_t2.astype(f32)
    b2 = jnp.zeros((8, NHID), f32).at[0].set(b_t2)
    wr_p = jnp.zeros((HP, HW), f32).at[:DREL].set(Wr)

    t_abs, tflat = _prep(pe_abs, w1, b1, w2, b2, pe_rel, wr_p)

    pos = jnp.arange(HWH)
    chan = (pos // 32) * 32 + (pos % 2) * L + (pos % 32) // 2
    base_lo = (chan // CH) * NHID + chan % CH
    cols_lo = base_lo
    cols_hi = base_lo + CH
    rows384 = jnp.arange(HWH) // CH
    mask64 = (rows384[:, None] == jnp.arange(LP)[None, :]).astype(f32)

    tflo = tflat[:, cols_lo]
    tfhi = tflat[:, cols_hi]

    def mk_ws(wg, a_s, a_d):
        wp = jnp.zeros((HP, HW), f32).at[:DIN].set(wg)
        asf = a_s.reshape(HW)
        adf = a_d.reshape(HW)

        def row8(v):
            return jnp.zeros((8, HWH), f32).at[0].set(v)
        return (wp[:, cols_lo], wp[:, cols_hi],
                row8(asf[cols_lo]), row8(asf[cols_hi]),
                row8(adf[cols_lo]), row8(adf[cols_hi]),
                tflo, tfhi, mask64)

    qpad = jnp.zeros((N, NHID), f32).at[:, :NLABEL].set(q_Y_sample)
    hpad0 = jnp.concatenate([x.astype(f32), qpad], axis=1)

    src = adj[0].astype(jnp.int32)
    dst = adj[1].astype(jnp.int32)
    src3 = src.reshape(NWORK, NCHUNK, K)
    dst3 = dst.reshape(NWORK, NCHUNK, K)
    src3b = src.reshape(NS, NCHUNK2, K2)
    dst3b = dst.reshape(NS, NCHUNK2, K2)

    bts = [jnp.zeros((8, NHID), f32).at[0].set(b)[0:1] + t_abs[0:1]
           for b in (bg0, bg1, bg2)]
    bts = [jnp.concatenate([b, jnp.zeros((7, NHID), f32)], axis=0)
           for b in bts]

    layers = [mk_ws(Wg0, as0, ad0), mk_ws(Wg1, as1, ad1), mk_ws(Wg2, as2, ad2)]

    u0 = u1 = None
    for li, ws in enumerate(layers):
        if li == 0:
            hwlo, hwhi, s1p, adp = _layer0(hpad0, *ws)
        else:
            hwlo, hwhi, s1p, adp = _layern(u0, u1, bts[li - 1], qpad, *ws)
        ex, den = _pass1(src3, dst3, s1p, adp)
        up = _pass2(src3b, dst3b, hwlo, hwhi, ex, den[0], den[1])
        u0, u1 = up[0], up[1]

    wf1 = jnp.zeros((HP, 384), f32).at[:FDIM, :2 * FDIM].set(Wf1)
    bf1p = jnp.zeros((8, 384), f32).at[0, :2 * FDIM].set(bf1)
    wf2 = jnp.zeros((384, NHID), f32).at[:2 * FDIM, :2 * NLABEL].set(Wf2)
    bf2p = jnp.zeros((8, NHID), f32).at[0, :2 * NLABEL].set(bf2)

    out = _final(u0, u1, bts[2], qpad, wf1, bf1p, wf2, bf2p)
    return out[:, :2 * NLABEL]


# speedup vs baseline: 58.2676x; 1.1384x over previous
import math

import jax
import jax.numpy as jnp
from jax import lax
from jax.experimental import pallas as pl
from jax.experimental.pallas import tpu as pltpu
from jax.experimental.pallas import tpu_sc as plsc

N = 10000
E = 320000
NFEAT = 128
NLABEL = 5
NHID = 128
NHEAD = 6
DIN = NFEAT + NLABEL
DREL = DIN + 1
FDIM = NHID + NLABEL
HW = NHEAD * NHID
HWH = HW // 2
CH = 64
HP = 256
LP = 16

NC, NS, L = 2, 16, 16
NWORK = NC * NS
EPW = E // NWORK
K = 80
NCHUNK = EPW // K
EPT = E // NS
K2 = 80
NCHUNK2 = EPT // K2
ALN = 624
TAIL = N - NS * ALN

BLK = 2000
GRID = N // BLK


def _prep_body(pe_abs, w1, b1, w2, b2, pe_rel, wr, t_abs_o, tflat_o):
    z = jnp.dot(pe_abs[...], w1[...], preferred_element_type=jnp.float32)
    z = z + b1[...]
    z = jnp.where(z > 0, z, jnp.exp(z) - 1.0)
    ta = jnp.dot(z, w2[...], preferred_element_type=jnp.float32) + b2[...]
    t_abs_o[...] = ta
    tflat_o[...] = jnp.dot(pe_rel[...], wr[...],
                           preferred_element_type=jnp.float32)


def _prep(pe_abs, w1, b1, w2, b2, pe_rel, wr):
    return pl.pallas_call(
        _prep_body,
        out_shape=(jax.ShapeDtypeStruct((8, NHID), jnp.float32),
                   jax.ShapeDtypeStruct((8, HW), jnp.float32)),
    )(pe_abs, w1, b1, w2, b2, pe_rel, wr)


def _matmul_tail(hblk, wlo, whi, aslo, ashi, adlo, adhi, tflo, tfhi, mask64,
                 lo_o, hi_o, s1_o, ad_o):
    lo = jnp.dot(hblk, wlo[...], preferred_element_type=jnp.float32)
    hi = jnp.dot(hblk, whi[...], preferred_element_type=jnp.float32)
    lo_o[...] = lo.astype(jnp.bfloat16)
    hi_o[...] = hi.astype(jnp.bfloat16)
    svlo = aslo[0:1, :] + tflo[0:1, :]
    svhi = ashi[0:1, :] + tfhi[0:1, :]
    s1_o[...] = (jnp.dot(lo * svlo, mask64[...],
                         preferred_element_type=jnp.float32) +
                 jnp.dot(hi * svhi, mask64[...],
                         preferred_element_type=jnp.float32))
    ad_o[...] = (jnp.dot(lo * adlo[0:1, :], mask64[...],
                         preferred_element_type=jnp.float32) +
                 jnp.dot(hi * adhi[0:1, :], mask64[...],
                         preferred_element_type=jnp.float32))


def _layer0_body(hpad, wlo, whi, aslo, ashi, adlo, adhi, tflo, tfhi, mask64,
                 lo_o, hi_o, s1_o, ad_o):
    _matmul_tail(hpad[...], wlo, whi, aslo, ashi, adlo, adhi, tflo, tfhi,
                 mask64, lo_o, hi_o, s1_o, ad_o)


def _layern_body(u0, u1, bt, qpad, wlo, whi, aslo, ashi, adlo, adhi,
                 tflo, tfhi, mask64, lo_o, hi_o, s1_o, ad_o):
    g = jnp.concatenate([u0[...], u1[...]], axis=1) * (1.0 / NHEAD)
    g = g + bt[0:1, :]
    e = jnp.where(g > 0, g, jnp.exp(g) - 1.0)
    hblk = jnp.concatenate([e, qpad[...]], axis=1)
    _matmul_tail(hblk, wlo, whi, aslo, ashi, adlo, adhi, tflo, tfhi,
                 mask64, lo_o, hi_o, s1_o, ad_o)


def _row_spec(w):
    return pl.BlockSpec((BLK, w), lambda i: (i, 0))


def _full_spec(r, c):
    return pl.BlockSpec((r, c), lambda i: (0, 0))


_LAYER_OUT = (jax.ShapeDtypeStruct((N, HWH), jnp.bfloat16),
              jax.ShapeDtypeStruct((N, HWH), jnp.bfloat16),
              jax.ShapeDtypeStruct((N, LP), jnp.float32),
              jax.ShapeDtypeStruct((N, LP), jnp.float32))
_LAYER_OUT_SPECS = (_row_spec(HWH), _row_spec(HWH),
                    _row_spec(LP), _row_spec(LP))
_WSPECS = [_full_spec(HP, HWH), _full_spec(HP, HWH), _full_spec(8, HWH),
           _full_spec(8, HWH), _full_spec(8, HWH), _full_spec(8, HWH),
           _full_spec(8, HWH), _full_spec(8, HWH), _full_spec(HWH, LP)]


def _layer0(hpad, *ws):
    return pl.pallas_call(
        _layer0_body,
        grid=(GRID,),
        in_specs=[_row_spec(HP)] + _WSPECS,
        out_specs=_LAYER_OUT_SPECS,
        out_shape=_LAYER_OUT,
    )(hpad, *ws)


def _layern(u0, u1, bt, qpad, *ws):
    return pl.pallas_call(
        _layern_body,
        grid=(GRID,),
        in_specs=[_row_spec(CH), _row_spec(CH), _full_spec(8, NHID),
                  _row_spec(NHID)] + _WSPECS,
        out_specs=_LAYER_OUT_SPECS,
        out_shape=_LAYER_OUT,
    )(u0, u1, bt, qpad, *ws)


def _recip_body(d0, d1, r_o):
    r_o[...] = 1.0 / (d0[...] + d1[...] + 1e-16)


def _recip(d0, d1):
    return pl.pallas_call(
        _recip_body,
        grid=(GRID,),
        in_specs=[_row_spec(LP), _row_spec(LP)],
        out_specs=_row_spec(LP),
        out_shape=jax.ShapeDtypeStruct((N, LP), jnp.float32),
    )(d0, d1)


def _final_body(u0, u1, bt, qpad, wf1, bf1, wf2, bf2, out_o):
    g = jnp.concatenate([u0[...], u1[...]], axis=1) * (1.0 / NHEAD)
    g = g + bt[0:1, :]
    e = jnp.where(g > 0, g, jnp.exp(g) - 1.0)
    hblk = jnp.concatenate([e, qpad[...]], axis=1)
    z = jnp.dot(hblk, wf1[...], preferred_element_type=jnp.float32)
    z = z + bf1[0:1, :]
    z = jnp.where(z > 0, z, jnp.exp(z) - 1.0)
    out_o[...] = jnp.dot(z, wf2[...],
                         preferred_element_type=jnp.float32) + bf2[0:1, :]


def _final(u0, u1, bt, qpad, wf1, bf1, wf2, bf2):
    return pl.pallas_call(
        _final_body,
        grid=(GRID,),
        in_specs=[_row_spec(CH), _row_spec(CH), _full_spec(8, NHID),
                  _row_spec(NHID), _full_spec(HP, 384), _full_spec(8, 384),
                  _full_spec(384, NHID), _full_spec(8, NHID)],
        out_specs=_row_spec(NHID),
        out_shape=jax.ShapeDtypeStruct((N, NHID), jnp.float32),
    )(u0, u1, bt, qpad, wf1, bf1, wf2, bf2)


_MESH = plsc.VectorSubcoreMesh(core_axis_name="c", subcore_axis_name="s",
                               num_cores=NC, num_subcores=NS)
_SC_PARAMS = pltpu.CompilerParams(use_tc_tiling_on_sc=False,
                                  needs_layout_passes=False)


def _pass1_body(src_h, dst_h, s1_h, ad_h, ex_h, den_h,
                src_pf, dst_pf, s1_a, s1_b, ad_a, ad_b, ex_v, zb, den_sp,
                sem_a, sem_b):
    cid = lax.axis_index("c")
    sid = lax.axis_index("s")
    wid = sid * NC + cid
    s1_v = (s1_a, s1_b)
    ad_v = (ad_a, ad_b)
    sem = (sem_a, sem_b)

    def zrow(i, _):
        zb[i, :] = jnp.zeros((L,), jnp.float32)
        return 0
    lax.fori_loop(0, ALN, zrow, 0)
    pltpu.sync_copy(zb, den_sp.at[pl.ds(sid * ALN, ALN)])

    @pl.when(sid == 0)
    def _():
        pltpu.sync_copy(zb.at[pl.ds(0, TAIL)],
                        den_sp.at[pl.ds(NS * ALN, TAIL)])

    pltpu.sync_copy(src_h.at[wid], src_pf)
    pltpu.sync_copy(dst_h.at[wid], dst_pf)
    plsc.subcore_barrier()

    def fire(c, b):
        pltpu.async_copy(s1_h.at[src_pf.at[c]], s1_v[b], sem[b])
        pltpu.async_copy(ad_h.at[dst_pf.at[c]], ad_v[b], sem[b])

    def drain(c, b):
        pltpu.make_async_copy(s1_h.at[src_pf.at[c]], s1_v[b], sem[b]).wait()
        pltpu.make_async_copy(ad_h.at[dst_pf.at[c]], ad_v[b], sem[b]).wait()

    def work(c, b):
        def edge(i, _):
            v = s1_v[b][i, :] + ad_v[b][i, :]
            v = jnp.where(v >= 0, v, v * 0.2)
            ex_v[i, :] = jnp.exp(v)
            return 0
        lax.fori_loop(0, K, edge, 0)
        pltpu.sync_copy(ex_v, den_sp.at[dst_pf.at[c]], add=True)
        pltpu.sync_copy(ex_v, ex_h.at[pl.ds(wid * EPW + c * K, K)])

    fire(0, 0)

    def pair(g, _):
        c0 = 2 * g
        fire(c0 + 1, 1)
        drain(c0, 0)
        work(c0, 0)
        fire(c0 + 2, 0)
        drain(c0 + 1, 1)
        work(c0 + 1, 1)
        return 0
    lax.fori_loop(0, (NCHUNK - 1) // 2, pair, 0)
    drain(NCHUNK - 1, 0)
    work(NCHUNK - 1, 0)

    plsc.subcore_barrier()
    pltpu.sync_copy(den_sp.at[pl.ds(sid * ALN, ALN)],
                    den_h.at[cid, pl.ds(sid * ALN, ALN)])

    @pl.when(sid == 0)
    def _():
        pltpu.sync_copy(den_sp.at[pl.ds(NS * ALN, TAIL)],
                        den_h.at[cid, pl.ds(NS * ALN, TAIL)])


def _pass1(src3, dst3, s1p, adp):
    return pl.kernel(
        _pass1_body,
        out_type=(jax.ShapeDtypeStruct((E, LP), jnp.float32),
                  jax.ShapeDtypeStruct((NC, N, LP), jnp.float32)),
        mesh=_MESH,
        scratch_types=[
            pltpu.VMEM((NCHUNK, K), jnp.int32),
            pltpu.VMEM((NCHUNK, K), jnp.int32),
            pltpu.VMEM((K, LP), jnp.float32),
            pltpu.VMEM((K, LP), jnp.float32),
            pltpu.VMEM((K, LP), jnp.float32),
            pltpu.VMEM((K, LP), jnp.float32),
            pltpu.VMEM((K, LP), jnp.float32),
            pltpu.VMEM((ALN, LP), jnp.float32),
            pltpu.VMEM_SHARED((N, LP), jnp.float32),
            pltpu.SemaphoreType.DMA,
            pltpu.SemaphoreType.DMA,
        ],
        compiler_params=_SC_PARAMS,
    )(src3, dst3, s1p, adp)


_ZROWS = 48


def _pass2_body(src_h, dst_h, lo_h, hi_h, ex_h, rec_h, up_h,
                src_pf, dst_pf, rows_a, rows_b, ex_a, ex_b,
                d0_a, d0_b, msg_v, zb, out_sp, sem_a, sem_b):
    cid = lax.axis_index("c")
    sid = lax.axis_index("s")
    lane = lax.broadcasted_iota(jnp.int32, (L,), 0)
    rows_v = (rows_a, rows_b)
    ex_v = (ex_a, ex_b)
    d0_v = (d0_a, d0_b)
    sem = (sem_a, sem_b)

    def zrow(i, _):
        for j in range(CH // L):
            zb[i, pl.ds(j * L, L)] = jnp.zeros((L,), jnp.float32)
        return 0
    lax.fori_loop(0, _ZROWS, zrow, 0)
    for kk in range(ALN // _ZROWS):
        pltpu.sync_copy(zb, out_sp.at[pl.ds(sid * ALN + kk * _ZROWS, _ZROWS)])

    @pl.when(sid == 0)
    def _():
        pltpu.sync_copy(zb.at[pl.ds(0, TAIL)],
                        out_sp.at[pl.ds(NS * ALN, TAIL)])

    pltpu.sync_copy(src_h.at[sid], src_pf)
    pltpu.sync_copy(dst_h.at[sid], dst_pf)
    plsc.subcore_barrier()

    def sweep(tab_h):
        def fire(c, b):
            pltpu.async_copy(tab_h.at[src_pf.at[c]], rows_v[b], sem[b])
            pltpu.async_copy(ex_h.at[pl.ds(sid * EPT + c * K2, K2)],
                             ex_v[b], sem[b])
            pltpu.async_copy(rec_h.at[dst_pf.at[c]], d0_v[b], sem[b])

        def drain(c, b):
            pltpu.make_async_copy(tab_h.at[src_pf.at[c]], rows_v[b],
                                  sem[b]).wait()
            pltpu.make_async_copy(ex_h.at[pl.ds(sid * EPT + c * K2, K2)],
                                  ex_v[b], sem[b]).wait()
            pltpu.make_async_copy(rec_h.at[dst_pf.at[c]], d0_v[b],
                                  sem[b]).wait()

        def work(c, b):
            def edge(i, _):
                w = ex_v[b][i, :] * d0_v[b][i, :]
                acc = [None] * (CH // L)
                for h in range(NHEAD):
                    wh = lax.gather(
                        w, (lane * 0 + h)[:, None],
                        lax.GatherDimensionNumbers(
                            offset_dims=(), collapsed_slice_dims=(0,),
                            start_index_map=(0,)),
                        (1,),
                        mode=lax.GatherScatterMode.PROMISE_IN_BOUNDS)
                    for g2 in range(CH // (2 * L)):
                        ab = rows_v[b][i, pl.ds(h * CH + g2 * 2 * L, 2 * L)]
                        pa, pb = plsc.unpack(
                            ab, format=plsc.PackFormat.INTERLEAVED,
                            preferred_element_type=jnp.float32)
                        j = 2 * g2
                        if h == 0:
                            acc[j] = wh * pa
                            acc[j + 1] = wh * pb
                        else:
                            acc[j] = acc[j] + wh * pa
                            acc[j + 1] = acc[j + 1] + wh * pb
                for j in range(CH // L):
                    msg_v[i, pl.ds(j * L, L)] = acc[j]
                return 0
            lax.fori_loop(0, K2, edge, 0)
            pltpu.sync_copy(msg_v, out_sp.at[dst_pf.at[c]], add=True)

        fire(0, 0)

        def pair(g, _):
            c0 = 2 * g
            fire(c0 + 1, 1)
            drain(c0, 0)
            work(c0, 0)
            fire(c0 + 2, 0)
            drain(c0 + 1, 1)
            work(c0 + 1, 1)
            return 0
        lax.fori_loop(0, (NCHUNK2 - 2) // 2, pair, 0)
        c0 = NCHUNK2 - 2
        fire(c0 + 1, 1)
        drain(c0, 0)
        work(c0, 0)
        drain(c0 + 1, 1)
        work(c0 + 1, 1)

    @pl.when(cid == 0)
    def _():
        sweep(lo_h)

    @pl.when(cid == 1)
    def _():
        sweep(hi_h)

    plsc.subcore_barrier()
    pltpu.sync_copy(out_sp.at[pl.ds(sid * ALN, ALN)],
                    up_h.at[cid, pl.ds(sid * ALN, ALN)])

    @pl.when(sid == 0)
    def _():
        pltpu.sync_copy(out_sp.at[pl.ds(NS * ALN, TAIL)],
                        up_h.at[cid, pl.ds(NS * ALN, TAIL)])


def _pass2(src3, dst3, hwlo, hwhi, ex, rec):
    return pl.kernel(
        _pass2_body,
        out_type=jax.ShapeDtypeStruct((NC, N, CH), jnp.float32),
        mesh=_MESH,
        scratch_types=[
            pltpu.VMEM((NCHUNK2, K2), jnp.int32),
            pltpu.VMEM((NCHUNK2, K2), jnp.int32),
            pltpu.VMEM((K2, HWH), jnp.bfloat16),
            pltpu.VMEM((K2, HWH), jnp.bfloat16),
            pltpu.VMEM((K2, LP), jnp.float32),
            pltpu.VMEM((K2, LP), jnp.float32),
            pltpu.VMEM((K2, LP), jnp.float32),
            pltpu.VMEM((K2, LP), jnp.float32),
            pltpu.VMEM((K2, CH), jnp.float32),
            pltpu.VMEM((_ZROWS, CH), jnp.float32),
            pltpu.VMEM_SHARED((N, CH), jnp.float32),
            pltpu.SemaphoreType.DMA,
            pltpu.SemaphoreType.DMA,
        ],
        compiler_params=_SC_PARAMS,
    )(src3, dst3, hwlo, hwhi, ex, rec)


def kernel(x, q_Y_sample, adj, t, num_steps, W_t1, b_t1, W_t2, b_t2, Wr,
           Wg0, as0, ad0, bg0, Wg1, as1, ad1, bg1, Wg2, as2, ad2, bg2,
           Wf1, bf1, Wf2, bf2):
    f32 = jnp.float32

    tv = t.astype(f32)
    half = NHID // 2
    emb = math.log(10000.0) / (half - 1)
    freqs = jnp.exp(jnp.arange(half, dtype=f32) * -emb)
    a = (tv * 4.0)[:, None] * freqs[None, :]
    pe_abs = jnp.concatenate([jnp.sin(a), jnp.cos(a)], axis=-1)
    pe_abs = jnp.zeros((8, NHID), f32).at[0:1].set(pe_abs)

    inv_freq = 1.0 / (10000.0 ** (jnp.arange(0.0, DIN, 2.0, dtype=f32) / DIN))
    si = tv[:, None] * inv_freq[None, :]
    pe_rel = jnp.concatenate([jnp.sin(si), jnp.cos(si)], axis=-1)
    pe_rel = jnp.zeros((8, HP), f32).at[0:1, :DREL].set(pe_rel)

    w1 = W_t1.astype(f32)
    b1 = jnp.zeros((8, 2 * NHID), f32).at[0].set(b_t1)
    w2 = W_t2.astype(f32)
    b2 = jnp.zeros((8, NHID), f32).at[0].set(b_t2)
    wr_p = jnp.zeros((HP, HW), f32).at[:DREL].set(Wr)

    t_abs, tflat = _prep(pe_abs, w1, b1, w2, b2, pe_rel, wr_p)

    pos = jnp.arange(HWH)
    chan = (pos // 32) * 32 + (pos % 2) * L + (pos % 32) // 2
    base_lo = (chan // CH) * NHID + chan % CH
    cols_lo = base_lo
    cols_hi = base_lo + CH
    rows384 = jnp.arange(HWH) // CH
    mask64 = (rows384[:, None] == jnp.arange(LP)[None, :]).astype(f32)

    tflo = tflat[:, cols_lo]
    tfhi = tflat[:, cols_hi]

    def mk_ws(wg, a_s, a_d):
        wp = jnp.zeros((HP, HW), f32).at[:DIN].set(wg)
        asf = a_s.reshape(HW)
        adf = a_d.reshape(HW)

        def row8(v):
            return jnp.zeros((8, HWH), f32).at[0].set(v)
        return (wp[:, cols_lo], wp[:, cols_hi],
                row8(asf[cols_lo]), row8(asf[cols_hi]),
                row8(adf[cols_lo]), row8(adf[cols_hi]),
                tflo, tfhi, mask64)

    qpad = jnp.zeros((N, NHID), f32).at[:, :NLABEL].set(q_Y_sample)
    hpad0 = jnp.concatenate([x.astype(f32), qpad], axis=1)

    src = adj[0].astype(jnp.int32)
    dst = adj[1].astype(jnp.int32)
    src3 = src.reshape(NWORK, NCHUNK, K)
    dst3 = dst.reshape(NWORK, NCHUNK, K)
    src3b = src.reshape(NS, NCHUNK2, K2)
    dst3b = dst.reshape(NS, NCHUNK2, K2)

    bts = [jnp.zeros((8, NHID), f32).at[0].set(b)[0:1] + t_abs[0:1]
           for b in (bg0, bg1, bg2)]
    bts = [jnp.concatenate([b, jnp.zeros((7, NHID), f32)], axis=0)
           for b in bts]

    layers = [mk_ws(Wg0, as0, ad0), mk_ws(Wg1, as1, ad1), mk_ws(Wg2, as2, ad2)]

    u0 = u1 = None
    for li, ws in enumerate(layers):
        if li == 0:
            hwlo, hwhi, s1p, adp = _layer0(hpad0, *ws)
        else:
            hwlo, hwhi, s1p, adp = _layern(u0, u1, bts[li - 1], qpad, *ws)
        ex, den = _pass1(src3, dst3, s1p, adp)
        rec = _recip(den[0], den[1])
        up = _pass2(src3b, dst3b, hwlo, hwhi, ex, rec)
        u0, u1 = up[0], up[1]

    wf1 = jnp.zeros((HP, 384), f32).at[:FDIM, :2 * FDIM].set(Wf1)
    bf1p = jnp.zeros((8, 384), f32).at[0, :2 * FDIM].set(bf1)
    wf2 = jnp.zeros((384, NHID), f32).at[:2 * FDIM, :2 * NLABEL].set(Wf2)
    bf2p = jnp.zeros((8, NHID), f32).at[0, :2 * NLABEL].set(bf2)

    out = _final(u0, u1, bts[2], qpad, wf1, bf1p, wf2, bf2p)
    return out[:, :2 * NLABEL]


# speedup vs baseline: 68.3233x; 1.1726x over previous
import math

import jax
import jax.numpy as jnp
from jax import lax
from jax.experimental import pallas as pl
from jax.experimental.pallas import tpu as pltpu
from jax.experimental.pallas import tpu_sc as plsc

N = 10000
E = 320000
NFEAT = 128
NLABEL = 5
NHID = 128
NHEAD = 6
DIN = NFEAT + NLABEL
DREL = DIN + 1
FDIM = NHID + NLABEL
HW = NHEAD * NHID
HWH = HW // 2
CH = 64
HP = 256
LP = 16

NC, NS, L = 2, 16, 16
NWORK = NC * NS
EPW = E // NWORK
K = 80
NCHUNK = EPW // K
EPT = E // NS
K2 = 80
NCHUNK2 = EPT // K2
ALN = 624
TAIL = N - NS * ALN

BLK = 2000
GRID = N // BLK


def _prep_body(pe_abs, w1, b1, w2, b2, pe_rel, wr, t_abs_o, tflat_o):
    z = jnp.dot(pe_abs[...], w1[...], preferred_element_type=jnp.float32)
    z = z + b1[...]
    z = jnp.where(z > 0, z, jnp.exp(z) - 1.0)
    ta = jnp.dot(z, w2[...], preferred_element_type=jnp.float32) + b2[...]
    t_abs_o[...] = ta
    tflat_o[...] = jnp.dot(pe_rel[...], wr[...],
                           preferred_element_type=jnp.float32)


def _prep(pe_abs, w1, b1, w2, b2, pe_rel, wr):
    return pl.pallas_call(
        _prep_body,
        out_shape=(jax.ShapeDtypeStruct((8, NHID), jnp.float32),
                   jax.ShapeDtypeStruct((8, HW), jnp.float32)),
    )(pe_abs, w1, b1, w2, b2, pe_rel, wr)


def _matmul_tail(hblk, wlo, whi, aslo, ashi, adlo, adhi, tflo, tfhi, mask64,
                 lo_o, hi_o, s1_o, ad_o):
    lo = jnp.dot(hblk, wlo[...], preferred_element_type=jnp.float32)
    hi = jnp.dot(hblk, whi[...], preferred_element_type=jnp.float32)
    lo_o[...] = lo.astype(jnp.bfloat16)
    hi_o[...] = hi.astype(jnp.bfloat16)
    svlo = aslo[0:1, :] + tflo[0:1, :]
    svhi = ashi[0:1, :] + tfhi[0:1, :]
    s1_o[...] = (jnp.dot(lo * svlo, mask64[...],
                         preferred_element_type=jnp.float32) +
                 jnp.dot(hi * svhi, mask64[...],
                         preferred_element_type=jnp.float32))
    ad_o[...] = (jnp.dot(lo * adlo[0:1, :], mask64[...],
                         preferred_element_type=jnp.float32) +
                 jnp.dot(hi * adhi[0:1, :], mask64[...],
                         preferred_element_type=jnp.float32))


def _layer0_body(hpad, wlo, whi, aslo, ashi, adlo, adhi, tflo, tfhi, mask64,
                 lo_o, hi_o, s1_o, ad_o):
    _matmul_tail(hpad[...], wlo, whi, aslo, ashi, adlo, adhi, tflo, tfhi,
                 mask64, lo_o, hi_o, s1_o, ad_o)


def _layern_body(u0, u1, bt, qpad, wlo, whi, aslo, ashi, adlo, adhi,
                 tflo, tfhi, mask64, lo_o, hi_o, s1_o, ad_o):
    g = jnp.concatenate([u0[...], u1[...]], axis=1) * (1.0 / NHEAD)
    g = g + bt[0:1, :]
    e = jnp.where(g > 0, g, jnp.exp(g) - 1.0)
    hblk = jnp.concatenate([e, qpad[...]], axis=1)
    _matmul_tail(hblk, wlo, whi, aslo, ashi, adlo, adhi, tflo, tfhi,
                 mask64, lo_o, hi_o, s1_o, ad_o)


def _row_spec(w):
    return pl.BlockSpec((BLK, w), lambda i: (i, 0))


def _full_spec(r, c):
    return pl.BlockSpec((r, c), lambda i: (0, 0))


_LAYER_OUT = (jax.ShapeDtypeStruct((N, HWH), jnp.bfloat16),
              jax.ShapeDtypeStruct((N, HWH), jnp.bfloat16),
              jax.ShapeDtypeStruct((N, LP), jnp.float32),
              jax.ShapeDtypeStruct((N, LP), jnp.float32))
_LAYER_OUT_SPECS = (_row_spec(HWH), _row_spec(HWH),
                    _row_spec(LP), _row_spec(LP))
_WSPECS = [_full_spec(HP, HWH), _full_spec(HP, HWH), _full_spec(8, HWH),
           _full_spec(8, HWH), _full_spec(8, HWH), _full_spec(8, HWH),
           _full_spec(8, HWH), _full_spec(8, HWH), _full_spec(HWH, LP)]


def _layer0(hpad, *ws):
    return pl.pallas_call(
        _layer0_body,
        grid=(GRID,),
        in_specs=[_row_spec(HP)] + _WSPECS,
        out_specs=_LAYER_OUT_SPECS,
        out_shape=_LAYER_OUT,
    )(hpad, *ws)


def _layern(u0, u1, bt, qpad, *ws):
    return pl.pallas_call(
        _layern_body,
        grid=(GRID,),
        in_specs=[_row_spec(CH), _row_spec(CH), _full_spec(8, NHID),
                  _row_spec(NHID)] + _WSPECS,
        out_specs=_LAYER_OUT_SPECS,
        out_shape=_LAYER_OUT,
    )(u0, u1, bt, qpad, *ws)


def _recip_body(d0, d1, r_o):
    r_o[...] = 1.0 / (d0[...] + d1[...] + 1e-16)


def _recip(d0, d1):
    return pl.pallas_call(
        _recip_body,
        grid=(GRID,),
        in_specs=[_row_spec(LP), _row_spec(LP)],
        out_specs=_row_spec(LP),
        out_shape=jax.ShapeDtypeStruct((N, LP), jnp.float32),
    )(d0, d1)


def _final_body(u0, u1, bt, qpad, wf1, bf1, wf2, bf2, out_o):
    g = jnp.concatenate([u0[...], u1[...]], axis=1) * (1.0 / NHEAD)
    g = g + bt[0:1, :]
    e = jnp.where(g > 0, g, jnp.exp(g) - 1.0)
    hblk = jnp.concatenate([e, qpad[...]], axis=1)
    z = jnp.dot(hblk, wf1[...], preferred_element_type=jnp.float32)
    z = z + bf1[0:1, :]
    z = jnp.where(z > 0, z, jnp.exp(z) - 1.0)
    out_o[...] = jnp.dot(z, wf2[...],
                         preferred_element_type=jnp.float32) + bf2[0:1, :]


def _final(u0, u1, bt, qpad, wf1, bf1, wf2, bf2):
    return pl.pallas_call(
        _final_body,
        grid=(GRID,),
        in_specs=[_row_spec(CH), _row_spec(CH), _full_spec(8, NHID),
                  _row_spec(NHID), _full_spec(HP, 384), _full_spec(8, 384),
                  _full_spec(384, NHID), _full_spec(8, NHID)],
        out_specs=_row_spec(NHID),
        out_shape=jax.ShapeDtypeStruct((N, NHID), jnp.float32),
    )(u0, u1, bt, qpad, wf1, bf1, wf2, bf2)


_MESH = plsc.VectorSubcoreMesh(core_axis_name="c", subcore_axis_name="s",
                               num_cores=NC, num_subcores=NS)
_SC_PARAMS = pltpu.CompilerParams(use_tc_tiling_on_sc=False,
                                  needs_layout_passes=False)


def _pass1_body(src_h, dst_h, s1_h, ad_h, ex_h, den_h,
                src_pf, dst_pf, s1_a, s1_b, ad_a, ad_b, ex_v, zb, den_sp,
                sem_a, sem_b):
    cid = lax.axis_index("c")
    sid = lax.axis_index("s")
    wid = sid * NC + cid
    s1_v = (s1_a, s1_b)
    ad_v = (ad_a, ad_b)
    sem = (sem_a, sem_b)

    def zrow(i, _):
        zb[i, :] = jnp.zeros((L,), jnp.float32)
        return 0
    lax.fori_loop(0, ALN, zrow, 0)
    pltpu.sync_copy(zb, den_sp.at[pl.ds(sid * ALN, ALN)])

    @pl.when(sid == 0)
    def _():
        pltpu.sync_copy(zb.at[pl.ds(0, TAIL)],
                        den_sp.at[pl.ds(NS * ALN, TAIL)])

    pltpu.sync_copy(src_h.at[wid], src_pf)
    pltpu.sync_copy(dst_h.at[wid], dst_pf)
    plsc.subcore_barrier()

    def fire(c, b):
        pltpu.async_copy(s1_h.at[src_pf.at[c]], s1_v[b], sem[b])
        pltpu.async_copy(ad_h.at[dst_pf.at[c]], ad_v[b], sem[b])

    def drain(c, b):
        pltpu.make_async_copy(s1_h.at[src_pf.at[c]], s1_v[b], sem[b]).wait()
        pltpu.make_async_copy(ad_h.at[dst_pf.at[c]], ad_v[b], sem[b]).wait()

    def work(c, b):
        def edge(i, _):
            v = s1_v[b][i, :] + ad_v[b][i, :]
            v = jnp.where(v >= 0, v, v * 0.2)
            ex_v[i, :] = jnp.exp(v)
            return 0
        lax.fori_loop(0, K, edge, 0)
        pltpu.sync_copy(ex_v, den_sp.at[dst_pf.at[c]], add=True)
        pltpu.sync_copy(ex_v, ex_h.at[pl.ds(wid * EPW + c * K, K)])

    fire(0, 0)

    def pair(g, _):
        c0 = 2 * g
        fire(c0 + 1, 1)
        drain(c0, 0)
        work(c0, 0)
        fire(c0 + 2, 0)
        drain(c0 + 1, 1)
        work(c0 + 1, 1)
        return 0
    lax.fori_loop(0, (NCHUNK - 1) // 2, pair, 0)
    drain(NCHUNK - 1, 0)
    work(NCHUNK - 1, 0)

    plsc.subcore_barrier()
    pltpu.sync_copy(den_sp.at[pl.ds(sid * ALN, ALN)],
                    den_h.at[cid, pl.ds(sid * ALN, ALN)])

    @pl.when(sid == 0)
    def _():
        pltpu.sync_copy(den_sp.at[pl.ds(NS * ALN, TAIL)],
                        den_h.at[cid, pl.ds(NS * ALN, TAIL)])


def _pass1(src3, dst3, s1p, adp):
    return pl.kernel(
        _pass1_body,
        out_type=(jax.ShapeDtypeStruct((E, LP), jnp.float32),
                  jax.ShapeDtypeStruct((NC, N, LP), jnp.float32)),
        mesh=_MESH,
        scratch_types=[
            pltpu.VMEM((NCHUNK, K), jnp.int32),
            pltpu.VMEM((NCHUNK, K), jnp.int32),
            pltpu.VMEM((K, LP), jnp.float32),
            pltpu.VMEM((K, LP), jnp.float32),
            pltpu.VMEM((K, LP), jnp.float32),
            pltpu.VMEM((K, LP), jnp.float32),
            pltpu.VMEM((K, LP), jnp.float32),
            pltpu.VMEM((ALN, LP), jnp.float32),
            pltpu.VMEM_SHARED((N, LP), jnp.float32),
            pltpu.SemaphoreType.DMA,
            pltpu.SemaphoreType.DMA,
        ],
        compiler_params=_SC_PARAMS,
    )(src3, dst3, s1p, adp)


_ZROWS = 48


def _pass2_body(src_h, dst_h, lo_h, hi_h, ex_h, rec_h, up_h,
                src_pf, dst_pf, rows_a, rows_b, ex_a, ex_b,
                d0_a, d0_b, msg_v, zb, out_sp, sem_a, sem_b):
    cid = lax.axis_index("c")
    sid = lax.axis_index("s")
    lane = lax.broadcasted_iota(jnp.int32, (L,), 0)
    rows_v = (rows_a, rows_b)
    ex_v = (ex_a, ex_b)
    d0_v = (d0_a, d0_b)
    sem = (sem_a, sem_b)

    def zrow(i, _):
        for j in range(CH // L):
            zb[i, pl.ds(j * L, L)] = jnp.zeros((L,), jnp.float32)
        return 0
    lax.fori_loop(0, _ZROWS, zrow, 0)
    for kk in range(ALN // _ZROWS):
        pltpu.sync_copy(zb, out_sp.at[pl.ds(sid * ALN + kk * _ZROWS, _ZROWS)])

    @pl.when(sid == 0)
    def _():
        pltpu.sync_copy(zb.at[pl.ds(0, TAIL)],
                        out_sp.at[pl.ds(NS * ALN, TAIL)])

    pltpu.sync_copy(src_h.at[sid], src_pf)
    pltpu.sync_copy(dst_h.at[sid], dst_pf)
    plsc.subcore_barrier()

    def sweep(tab_h):
        def fire(c, b):
            pltpu.async_copy(tab_h.at[src_pf.at[c]], rows_v[b], sem[b])
            pltpu.async_copy(ex_h.at[pl.ds(sid * EPT + c * K2, K2)],
                             ex_v[b], sem[b])
            pltpu.async_copy(rec_h.at[dst_pf.at[c]], d0_v[b], sem[b])

        def drain(c, b):
            pltpu.make_async_copy(tab_h.at[src_pf.at[c]], rows_v[b],
                                  sem[b]).wait()
            pltpu.make_async_copy(ex_h.at[pl.ds(sid * EPT + c * K2, K2)],
                                  ex_v[b], sem[b]).wait()
            pltpu.make_async_copy(rec_h.at[dst_pf.at[c]], d0_v[b],
                                  sem[b]).wait()

        def work(c, b):
            @plsc.parallel_loop(0, K2, unroll=4)
            def edge(i):
                w = ex_v[b][i, :] * d0_v[b][i, :]
                acc = [None] * (CH // L)
                for h in range(NHEAD):
                    wh = lax.gather(
                        w, (lane * 0 + h)[:, None],
                        lax.GatherDimensionNumbers(
                            offset_dims=(), collapsed_slice_dims=(0,),
                            start_index_map=(0,)),
                        (1,),
                        mode=lax.GatherScatterMode.PROMISE_IN_BOUNDS)
                    for g2 in range(CH // (2 * L)):
                        ab = rows_v[b][i, pl.ds(h * CH + g2 * 2 * L, 2 * L)]
                        pa, pb = plsc.unpack(
                            ab, format=plsc.PackFormat.INTERLEAVED,
                            preferred_element_type=jnp.float32)
                        j = 2 * g2
                        if h == 0:
                            acc[j] = wh * pa
                            acc[j + 1] = wh * pb
                        else:
                            acc[j] = acc[j] + wh * pa
                            acc[j + 1] = acc[j + 1] + wh * pb
                for j in range(CH // L):
                    msg_v[i, pl.ds(j * L, L)] = acc[j]
            pltpu.sync_copy(msg_v, out_sp.at[dst_pf.at[c]], add=True)

        fire(0, 0)

        def pair(g, _):
            c0 = 2 * g
            fire(c0 + 1, 1)
            drain(c0, 0)
            work(c0, 0)
            fire(c0 + 2, 0)
            drain(c0 + 1, 1)
            work(c0 + 1, 1)
            return 0
        lax.fori_loop(0, (NCHUNK2 - 2) // 2, pair, 0)
        c0 = NCHUNK2 - 2
        fire(c0 + 1, 1)
        drain(c0, 0)
        work(c0, 0)
        drain(c0 + 1, 1)
        work(c0 + 1, 1)

    @pl.when(cid == 0)
    def _():
        sweep(lo_h)

    @pl.when(cid == 1)
    def _():
        sweep(hi_h)

    plsc.subcore_barrier()
    pltpu.sync_copy(out_sp.at[pl.ds(sid * ALN, ALN)],
                    up_h.at[cid, pl.ds(sid * ALN, ALN)])

    @pl.when(sid == 0)
    def _():
        pltpu.sync_copy(out_sp.at[pl.ds(NS * ALN, TAIL)],
                        up_h.at[cid, pl.ds(NS * ALN, TAIL)])


def _pass2(src3, dst3, hwlo, hwhi, ex, rec):
    return pl.kernel(
        _pass2_body,
        out_type=jax.ShapeDtypeStruct((NC, N, CH), jnp.float32),
        mesh=_MESH,
        scratch_types=[
            pltpu.VMEM((NCHUNK2, K2), jnp.int32),
            pltpu.VMEM((NCHUNK2, K2), jnp.int32),
            pltpu.VMEM((K2, HWH), jnp.bfloat16),
            pltpu.VMEM((K2, HWH), jnp.bfloat16),
            pltpu.VMEM((K2, LP), jnp.float32),
            pltpu.VMEM((K2, LP), jnp.float32),
            pltpu.VMEM((K2, LP), jnp.float32),
            pltpu.VMEM((K2, LP), jnp.float32),
            pltpu.VMEM((K2, CH), jnp.float32),
            pltpu.VMEM((_ZROWS, CH), jnp.float32),
            pltpu.VMEM_SHARED((N, CH), jnp.float32),
            pltpu.SemaphoreType.DMA,
            pltpu.SemaphoreType.DMA,
        ],
        compiler_params=_SC_PARAMS,
    )(src3, dst3, hwlo, hwhi, ex, rec)


def kernel(x, q_Y_sample, adj, t, num_steps, W_t1, b_t1, W_t2, b_t2, Wr,
           Wg0, as0, ad0, bg0, Wg1, as1, ad1, bg1, Wg2, as2, ad2, bg2,
           Wf1, bf1, Wf2, bf2):
    f32 = jnp.float32

    tv = t.astype(f32)
    half = NHID // 2
    emb = math.log(10000.0) / (half - 1)
    freqs = jnp.exp(jnp.arange(half, dtype=f32) * -emb)
    a = (tv * 4.0)[:, None] * freqs[None, :]
    pe_abs = jnp.concatenate([jnp.sin(a), jnp.cos(a)], axis=-1)
    pe_abs = jnp.zeros((8, NHID), f32).at[0:1].set(pe_abs)

    inv_freq = 1.0 / (10000.0 ** (jnp.arange(0.0, DIN, 2.0, dtype=f32) / DIN))
    si = tv[:, None] * inv_freq[None, :]
    pe_rel = jnp.concatenate([jnp.sin(si), jnp.cos(si)], axis=-1)
    pe_rel = jnp.zeros((8, HP), f32).at[0:1, :DREL].set(pe_rel)

    w1 = W_t1.astype(f32)
    b1 = jnp.zeros((8, 2 * NHID), f32).at[0].set(b_t1)
    w2 = W_t2.astype(f32)
    b2 = jnp.zeros((8, NHID), f32).at[0].set(b_t2)
    wr_p = jnp.zeros((HP, HW), f32).at[:DREL].set(Wr)

    t_abs, tflat = _prep(pe_abs, w1, b1, w2, b2, pe_rel, wr_p)

    pos = jnp.arange(HWH)
    chan = (pos // 32) * 32 + (pos % 2) * L + (pos % 32) // 2
    base_lo = (chan // CH) * NHID + chan % CH
    cols_lo = base_lo
    cols_hi = base_lo + CH
    rows384 = jnp.arange(HWH) // CH
    mask64 = (rows384[:, None] == jnp.arange(LP)[None, :]).astype(f32)

    tflo = tflat[:, cols_lo]
    tfhi = tflat[:, cols_hi]

    def mk_ws(wg, a_s, a_d):
        wp = jnp.zeros((HP, HW), f32).at[:DIN].set(wg)
        asf = a_s.reshape(HW)
        adf = a_d.reshape(HW)

        def row8(v):
            return jnp.zeros((8, HWH), f32).at[0].set(v)
        return (wp[:, cols_lo], wp[:, cols_hi],
                row8(asf[cols_lo]), row8(asf[cols_hi]),
                row8(adf[cols_lo]), row8(adf[cols_hi]),
                tflo, tfhi, mask64)

    qpad = jnp.zeros((N, NHID), f32).at[:, :NLABEL].set(q_Y_sample)
    hpad0 = jnp.concatenate([x.astype(f32), qpad], axis=1)

    src = adj[0].astype(jnp.int32)
    dst = adj[1].astype(jnp.int32)
    src3 = src.reshape(NWORK, NCHUNK, K)
    dst3 = dst.reshape(NWORK, NCHUNK, K)
    src3b = src.reshape(NS, NCHUNK2, K2)
    dst3b = dst.reshape(NS, NCHUNK2, K2)

    bts = [jnp.zeros((8, NHID), f32).at[0].set(b)[0:1] + t_abs[0:1]
           for b in (bg0, bg1, bg2)]
    bts = [jnp.concatenate([b, jnp.zeros((7, NHID), f32)], axis=0)
           for b in bts]

    layers = [mk_ws(Wg0, as0, ad0), mk_ws(Wg1, as1, ad1), mk_ws(Wg2, as2, ad2)]

    u0 = u1 = None
    for li, ws in enumerate(layers):
        if li == 0:
            hwlo, hwhi, s1p, adp = _layer0(hpad0, *ws)
        else:
            hwlo, hwhi, s1p, adp = _layern(u0, u1, bts[li - 1], qpad, *ws)
        ex, den = _pass1(src3, dst3, s1p, adp)
        rec = _recip(den[0], den[1])
        up = _pass2(src3b, dst3b, hwlo, hwhi, ex, rec)
        u0, u1 = up[0], up[1]

    wf1 = jnp.zeros((HP, 384), f32).at[:FDIM, :2 * FDIM].set(Wf1)
    bf1p = jnp.zeros((8, 384), f32).at[0, :2 * FDIM].set(bf1)
    wf2 = jnp.zeros((384, NHID), f32).at[:2 * FDIM, :2 * NLABEL].set(Wf2)
    bf2p = jnp.zeros((8, NHID), f32).at[0, :2 * NLABEL].set(bf2)

    out = _final(u0, u1, bts[2], qpad, wf1, bf1p, wf2, bf2p)
    return out[:, :2 * NLABEL]


# speedup vs baseline: 69.6928x; 1.0200x over previous
import math

import jax
import jax.numpy as jnp
from jax import lax
from jax.experimental import pallas as pl
from jax.experimental.pallas import tpu as pltpu
from jax.experimental.pallas import tpu_sc as plsc

N = 10000
E = 320000
NFEAT = 128
NLABEL = 5
NHID = 128
NHEAD = 6
DIN = NFEAT + NLABEL
DREL = DIN + 1
FDIM = NHID + NLABEL
HW = NHEAD * NHID
HWH = HW // 2
CH = 64
HP = 256
LP = 16

NC, NS, L = 2, 16, 16
NWORK = NC * NS
EPW = E // NWORK
K = 80
NCHUNK = EPW // K
EPT = E // NS
K2 = 80
NCHUNK2 = EPT // K2
ALN = 624
TAIL = N - NS * ALN

BLK = 2000
GRID = N // BLK


def _prep_body(pe_abs, w1, b1, w2, b2, pe_rel, wr, t_abs_o, tflat_o):
    z = jnp.dot(pe_abs[...], w1[...], preferred_element_type=jnp.float32)
    z = z + b1[...]
    z = jnp.where(z > 0, z, jnp.exp(z) - 1.0)
    ta = jnp.dot(z, w2[...], preferred_element_type=jnp.float32) + b2[...]
    t_abs_o[...] = ta
    tflat_o[...] = jnp.dot(pe_rel[...], wr[...],
                           preferred_element_type=jnp.float32)


def _prep(pe_abs, w1, b1, w2, b2, pe_rel, wr):
    return pl.pallas_call(
        _prep_body,
        out_shape=(jax.ShapeDtypeStruct((8, NHID), jnp.float32),
                   jax.ShapeDtypeStruct((8, HW), jnp.float32)),
    )(pe_abs, w1, b1, w2, b2, pe_rel, wr)


def _matmul_tail(hblk, wlo, whi, aslo, ashi, adlo, adhi, tflo, tfhi, mask64,
                 lo_o, hi_o, s1_o, ad_o):
    lo = jnp.dot(hblk, wlo[...], preferred_element_type=jnp.float32)
    hi = jnp.dot(hblk, whi[...], preferred_element_type=jnp.float32)
    lo_o[...] = lo.astype(jnp.bfloat16)
    hi_o[...] = hi.astype(jnp.bfloat16)
    svlo = aslo[0:1, :] + tflo[0:1, :]
    svhi = ashi[0:1, :] + tfhi[0:1, :]
    s1_o[...] = (jnp.dot(lo * svlo, mask64[...],
                         preferred_element_type=jnp.float32) +
                 jnp.dot(hi * svhi, mask64[...],
                         preferred_element_type=jnp.float32))
    ad_o[...] = (jnp.dot(lo * adlo[0:1, :], mask64[...],
                         preferred_element_type=jnp.float32) +
                 jnp.dot(hi * adhi[0:1, :], mask64[...],
                         preferred_element_type=jnp.float32))


def _layer0_body(hpad, wlo, whi, aslo, ashi, adlo, adhi, tflo, tfhi, mask64,
                 lo_o, hi_o, s1_o, ad_o):
    _matmul_tail(hpad[...], wlo, whi, aslo, ashi, adlo, adhi, tflo, tfhi,
                 mask64, lo_o, hi_o, s1_o, ad_o)


def _layern_body(u0, u1, bt, qpad, wlo, whi, aslo, ashi, adlo, adhi,
                 tflo, tfhi, mask64, lo_o, hi_o, s1_o, ad_o):
    g = jnp.concatenate([u0[...], u1[...]], axis=1) * (1.0 / NHEAD)
    g = g + bt[0:1, :]
    e = jnp.where(g > 0, g, jnp.exp(g) - 1.0)
    hblk = jnp.concatenate([e, qpad[...]], axis=1)
    _matmul_tail(hblk, wlo, whi, aslo, ashi, adlo, adhi, tflo, tfhi,
                 mask64, lo_o, hi_o, s1_o, ad_o)


def _row_spec(w):
    return pl.BlockSpec((BLK, w), lambda i: (i, 0))


def _full_spec(r, c):
    return pl.BlockSpec((r, c), lambda i: (0, 0))


_LAYER_OUT = (jax.ShapeDtypeStruct((N, HWH), jnp.bfloat16),
              jax.ShapeDtypeStruct((N, HWH), jnp.bfloat16),
              jax.ShapeDtypeStruct((N, LP), jnp.float32),
              jax.ShapeDtypeStruct((N, LP), jnp.float32))
_LAYER_OUT_SPECS = (_row_spec(HWH), _row_spec(HWH),
                    _row_spec(LP), _row_spec(LP))
_WSPECS = [_full_spec(HP, HWH), _full_spec(HP, HWH), _full_spec(8, HWH),
           _full_spec(8, HWH), _full_spec(8, HWH), _full_spec(8, HWH),
           _full_spec(8, HWH), _full_spec(8, HWH), _full_spec(HWH, LP)]


def _layer0(hpad, *ws):
    return pl.pallas_call(
        _layer0_body,
        grid=(GRID,),
        in_specs=[_row_spec(HP)] + _WSPECS,
        out_specs=_LAYER_OUT_SPECS,
        out_shape=_LAYER_OUT,
    )(hpad, *ws)


def _layern(u0, u1, bt, qpad, *ws):
    return pl.pallas_call(
        _layern_body,
        grid=(GRID,),
        in_specs=[_row_spec(CH), _row_spec(CH), _full_spec(8, NHID),
                  _row_spec(NHID)] + _WSPECS,
        out_specs=_LAYER_OUT_SPECS,
        out_shape=_LAYER_OUT,
    )(u0, u1, bt, qpad, *ws)


def _recip_body(d0, d1, r_o):
    r_o[...] = 1.0 / (d0[...] + d1[...] + 1e-16)


def _recip(d0, d1):
    return pl.pallas_call(
        _recip_body,
        grid=(GRID,),
        in_specs=[_row_spec(LP), _row_spec(LP)],
        out_specs=_row_spec(LP),
        out_shape=jax.ShapeDtypeStruct((N, LP), jnp.float32),
    )(d0, d1)


def _final_body(u0, u1, bt, qpad, wf1, bf1, wf2, bf2, out_o):
    g = jnp.concatenate([u0[...], u1[...]], axis=1) * (1.0 / NHEAD)
    g = g + bt[0:1, :]
    e = jnp.where(g > 0, g, jnp.exp(g) - 1.0)
    hblk = jnp.concatenate([e, qpad[...]], axis=1)
    z = jnp.dot(hblk, wf1[...], preferred_element_type=jnp.float32)
    z = z + bf1[0:1, :]
    z = jnp.where(z > 0, z, jnp.exp(z) - 1.0)
    out_o[...] = jnp.dot(z, wf2[...],
                         preferred_element_type=jnp.float32) + bf2[0:1, :]


def _final(u0, u1, bt, qpad, wf1, bf1, wf2, bf2):
    return pl.pallas_call(
        _final_body,
        grid=(GRID,),
        in_specs=[_row_spec(CH), _row_spec(CH), _full_spec(8, NHID),
                  _row_spec(NHID), _full_spec(HP, 384), _full_spec(8, 384),
                  _full_spec(384, NHID), _full_spec(8, NHID)],
        out_specs=_row_spec(NHID),
        out_shape=jax.ShapeDtypeStruct((N, NHID), jnp.float32),
    )(u0, u1, bt, qpad, wf1, bf1, wf2, bf2)


_MESH = plsc.VectorSubcoreMesh(core_axis_name="c", subcore_axis_name="s",
                               num_cores=NC, num_subcores=NS)
_SC_PARAMS = pltpu.CompilerParams(use_tc_tiling_on_sc=False,
                                  needs_layout_passes=False)


def _pass1_body(src_h, dst_h, s1_h, ad_h, ex_h, den_h,
                src_pf, dst_pf, s1_a, s1_b, ad_a, ad_b, ex_v, zb, den_sp,
                sem_a, sem_b):
    cid = lax.axis_index("c")
    sid = lax.axis_index("s")
    wid = sid * NC + cid
    s1_v = (s1_a, s1_b)
    ad_v = (ad_a, ad_b)
    sem = (sem_a, sem_b)

    def zrow(i, _):
        zb[i, :] = jnp.zeros((L,), jnp.float32)
        return 0
    lax.fori_loop(0, ALN, zrow, 0)
    pltpu.sync_copy(zb, den_sp.at[pl.ds(sid * ALN, ALN)])

    @pl.when(sid == 0)
    def _():
        pltpu.sync_copy(zb.at[pl.ds(0, TAIL)],
                        den_sp.at[pl.ds(NS * ALN, TAIL)])

    pltpu.sync_copy(src_h.at[wid], src_pf)
    pltpu.sync_copy(dst_h.at[wid], dst_pf)
    plsc.subcore_barrier()

    def fire(c, b):
        pltpu.async_copy(s1_h.at[src_pf.at[c]], s1_v[b], sem[b])
        pltpu.async_copy(ad_h.at[dst_pf.at[c]], ad_v[b], sem[b])

    def drain(c, b):
        pltpu.make_async_copy(s1_h.at[src_pf.at[c]], s1_v[b], sem[b]).wait()
        pltpu.make_async_copy(ad_h.at[dst_pf.at[c]], ad_v[b], sem[b]).wait()

    def work(c, b):
        @plsc.parallel_loop(0, K, unroll=8)
        def edge(i):
            v = s1_v[b][i, :] + ad_v[b][i, :]
            v = jnp.where(v >= 0, v, v * 0.2)
            ex_v[i, :] = jnp.exp(v)
        pltpu.sync_copy(ex_v, den_sp.at[dst_pf.at[c]], add=True)
        pltpu.sync_copy(ex_v, ex_h.at[pl.ds(wid * EPW + c * K, K)])

    fire(0, 0)

    def pair(g, _):
        c0 = 2 * g
        fire(c0 + 1, 1)
        drain(c0, 0)
        work(c0, 0)
        fire(c0 + 2, 0)
        drain(c0 + 1, 1)
        work(c0 + 1, 1)
        return 0
    lax.fori_loop(0, (NCHUNK - 1) // 2, pair, 0)
    drain(NCHUNK - 1, 0)
    work(NCHUNK - 1, 0)

    plsc.subcore_barrier()
    pltpu.sync_copy(den_sp.at[pl.ds(sid * ALN, ALN)],
                    den_h.at[cid, pl.ds(sid * ALN, ALN)])

    @pl.when(sid == 0)
    def _():
        pltpu.sync_copy(den_sp.at[pl.ds(NS * ALN, TAIL)],
                        den_h.at[cid, pl.ds(NS * ALN, TAIL)])


def _pass1(src3, dst3, s1p, adp):
    return pl.kernel(
        _pass1_body,
        out_type=(jax.ShapeDtypeStruct((E, LP), jnp.float32),
                  jax.ShapeDtypeStruct((NC, N, LP), jnp.float32)),
        mesh=_MESH,
        scratch_types=[
            pltpu.VMEM((NCHUNK, K), jnp.int32),
            pltpu.VMEM((NCHUNK, K), jnp.int32),
            pltpu.VMEM((K, LP), jnp.float32),
            pltpu.VMEM((K, LP), jnp.float32),
            pltpu.VMEM((K, LP), jnp.float32),
            pltpu.VMEM((K, LP), jnp.float32),
            pltpu.VMEM((K, LP), jnp.float32),
            pltpu.VMEM((ALN, LP), jnp.float32),
            pltpu.VMEM_SHARED((N, LP), jnp.float32),
            pltpu.SemaphoreType.DMA,
            pltpu.SemaphoreType.DMA,
        ],
        compiler_params=_SC_PARAMS,
    )(src3, dst3, s1p, adp)


_ZROWS = 48


def _pass2_body(src_h, dst_h, lo_h, hi_h, ex_h, rec_h, up_h,
                src_pf, dst_pf, rows_a, rows_b, ex_a, ex_b,
                d0_a, d0_b, msg_v, zb, out_sp, sem_a, sem_b):
    cid = lax.axis_index("c")
    sid = lax.axis_index("s")
    lane = lax.broadcasted_iota(jnp.int32, (L,), 0)
    rows_v = (rows_a, rows_b)
    ex_v = (ex_a, ex_b)
    d0_v = (d0_a, d0_b)
    sem = (sem_a, sem_b)

    def zrow(i, _):
        for j in range(CH // L):
            zb[i, pl.ds(j * L, L)] = jnp.zeros((L,), jnp.float32)
        return 0
    lax.fori_loop(0, _ZROWS, zrow, 0)
    for kk in range(ALN // _ZROWS):
        pltpu.sync_copy(zb, out_sp.at[pl.ds(sid * ALN + kk * _ZROWS, _ZROWS)])

    @pl.when(sid == 0)
    def _():
        pltpu.sync_copy(zb.at[pl.ds(0, TAIL)],
                        out_sp.at[pl.ds(NS * ALN, TAIL)])

    pltpu.sync_copy(src_h.at[sid], src_pf)
    pltpu.sync_copy(dst_h.at[sid], dst_pf)
    plsc.subcore_barrier()

    def sweep(tab_h):
        def fire(c, b):
            pltpu.async_copy(tab_h.at[src_pf.at[c]], rows_v[b], sem[b])
            pltpu.async_copy(ex_h.at[pl.ds(sid * EPT + c * K2, K2)],
                             ex_v[b], sem[b])
            pltpu.async_copy(rec_h.at[dst_pf.at[c]], d0_v[b], sem[b])

        def drain(c, b):
            pltpu.make_async_copy(tab_h.at[src_pf.at[c]], rows_v[b],
                                  sem[b]).wait()
            pltpu.make_async_copy(ex_h.at[pl.ds(sid * EPT + c * K2, K2)],
                                  ex_v[b], sem[b]).wait()
            pltpu.make_async_copy(rec_h.at[dst_pf.at[c]], d0_v[b],
                                  sem[b]).wait()

        def work(c, b):
            @plsc.parallel_loop(0, K2, unroll=4)
            def edge(i):
                w = ex_v[b][i, :] * d0_v[b][i, :]
                acc = [None] * (CH // L)
                for h in range(NHEAD):
                    wh = lax.gather(
                        w, (lane * 0 + h)[:, None],
                        lax.GatherDimensionNumbers(
                            offset_dims=(), collapsed_slice_dims=(0,),
                            start_index_map=(0,)),
                        (1,),
                        mode=lax.GatherScatterMode.PROMISE_IN_BOUNDS)
                    for g2 in range(CH // (2 * L)):
                        ab = rows_v[b][i, pl.ds(h * CH + g2 * 2 * L, 2 * L)]
                        pa, pb = plsc.unpack(
                            ab, format=plsc.PackFormat.INTERLEAVED,
                            preferred_element_type=jnp.float32)
                        j = 2 * g2
                        if h == 0:
                            acc[j] = wh * pa
                            acc[j + 1] = wh * pb
                        else:
                            acc[j] = acc[j] + wh * pa
                            acc[j + 1] = acc[j + 1] + wh * pb
                for j in range(CH // L):
                    msg_v[i, pl.ds(j * L, L)] = acc[j]
            pltpu.sync_copy(msg_v, out_sp.at[dst_pf.at[c]], add=True)

        fire(0, 0)

        def pair(g, _):
            c0 = 2 * g
            fire(c0 + 1, 1)
            drain(c0, 0)
            work(c0, 0)
            fire(c0 + 2, 0)
            drain(c0 + 1, 1)
            work(c0 + 1, 1)
            return 0
        lax.fori_loop(0, (NCHUNK2 - 2) // 2, pair, 0)
        c0 = NCHUNK2 - 2
        fire(c0 + 1, 1)
        drain(c0, 0)
        work(c0, 0)
        drain(c0 + 1, 1)
        work(c0 + 1, 1)

    @pl.when(cid == 0)
    def _():
        sweep(lo_h)

    @pl.when(cid == 1)
    def _():
        sweep(hi_h)

    plsc.subcore_barrier()
    pltpu.sync_copy(out_sp.at[pl.ds(sid * ALN, ALN)],
                    up_h.at[cid, pl.ds(sid * ALN, ALN)])

    @pl.when(sid == 0)
    def _():
        pltpu.sync_copy(out_sp.at[pl.ds(NS * ALN, TAIL)],
                        up_h.at[cid, pl.ds(NS * ALN, TAIL)])


def _pass2(src3, dst3, hwlo, hwhi, ex, rec):
    return pl.kernel(
        _pass2_body,
        out_type=jax.ShapeDtypeStruct((NC, N, CH), jnp.float32),
        mesh=_MESH,
        scratch_types=[
            pltpu.VMEM((NCHUNK2, K2), jnp.int32),
            pltpu.VMEM((NCHUNK2, K2), jnp.int32),
            pltpu.VMEM((K2, HWH), jnp.bfloat16),
            pltpu.VMEM((K2, HWH), jnp.bfloat16),
            pltpu.VMEM((K2, LP), jnp.float32),
            pltpu.VMEM((K2, LP), jnp.float32),
            pltpu.VMEM((K2, LP), jnp.float32),
            pltpu.VMEM((K2, LP), jnp.float32),
            pltpu.VMEM((K2, CH), jnp.float32),
            pltpu.VMEM((_ZROWS, CH), jnp.float32),
            pltpu.VMEM_SHARED((N, CH), jnp.float32),
            pltpu.SemaphoreType.DMA,
            pltpu.SemaphoreType.DMA,
        ],
        compiler_params=_SC_PARAMS,
    )(src3, dst3, hwlo, hwhi, ex, rec)


def kernel(x, q_Y_sample, adj, t, num_steps, W_t1, b_t1, W_t2, b_t2, Wr,
           Wg0, as0, ad0, bg0, Wg1, as1, ad1, bg1, Wg2, as2, ad2, bg2,
           Wf1, bf1, Wf2, bf2):
    f32 = jnp.float32

    tv = t.astype(f32)
    half = NHID // 2
    emb = math.log(10000.0) / (half - 1)
    freqs = jnp.exp(jnp.arange(half, dtype=f32) * -emb)
    a = (tv * 4.0)[:, None] * freqs[None, :]
    pe_abs = jnp.concatenate([jnp.sin(a), jnp.cos(a)], axis=-1)
    pe_abs = jnp.zeros((8, NHID), f32).at[0:1].set(pe_abs)

    inv_freq = 1.0 / (10000.0 ** (jnp.arange(0.0, DIN, 2.0, dtype=f32) / DIN))
    si = tv[:, None] * inv_freq[None, :]
    pe_rel = jnp.concatenate([jnp.sin(si), jnp.cos(si)], axis=-1)
    pe_rel = jnp.zeros((8, HP), f32).at[0:1, :DREL].set(pe_rel)

    w1 = W_t1.astype(f32)
    b1 = jnp.zeros((8, 2 * NHID), f32).at[0].set(b_t1)
    w2 = W_t2.astype(f32)
    b2 = jnp.zeros((8, NHID), f32).at[0].set(b_t2)
    wr_p = jnp.zeros((HP, HW), f32).at[:DREL].set(Wr)

    t_abs, tflat = _prep(pe_abs, w1, b1, w2, b2, pe_rel, wr_p)

    pos = jnp.arange(HWH)
    chan = (pos // 32) * 32 + (pos % 2) * L + (pos % 32) // 2
    base_lo = (chan // CH) * NHID + chan % CH
    cols_lo = base_lo
    cols_hi = base_lo + CH
    rows384 = jnp.arange(HWH) // CH
    mask64 = (rows384[:, None] == jnp.arange(LP)[None, :]).astype(f32)

    tflo = tflat[:, cols_lo]
    tfhi = tflat[:, cols_hi]

    def mk_ws(wg, a_s, a_d):
        wp = jnp.zeros((HP, HW), f32).at[:DIN].set(wg)
        asf = a_s.reshape(HW)
        adf = a_d.reshape(HW)

        def row8(v):
            return jnp.zeros((8, HWH), f32).at[0].set(v)
        return (wp[:, cols_lo], wp[:, cols_hi],
                row8(asf[cols_lo]), row8(asf[cols_hi]),
                row8(adf[cols_lo]), row8(adf[cols_hi]),
                tflo, tfhi, mask64)

    qpad = jnp.zeros((N, NHID), f32).at[:, :NLABEL].set(q_Y_sample)
    hpad0 = jnp.concatenate([x.astype(f32), qpad], axis=1)

    src = adj[0].astype(jnp.int32)
    dst = adj[1].astype(jnp.int32)
    src3 = src.reshape(NWORK, NCHUNK, K)
    dst3 = dst.reshape(NWORK, NCHUNK, K)
    src3b = src.reshape(NS, NCHUNK2, K2)
    dst3b = dst.reshape(NS, NCHUNK2, K2)

    bts = [jnp.zeros((8, NHID), f32).at[0].set(b)[0:1] + t_abs[0:1]
           for b in (bg0, bg1, bg2)]
    bts = [jnp.concatenate([b, jnp.zeros((7, NHID), f32)], axis=0)
           for b in bts]

    layers = [mk_ws(Wg0, as0, ad0), mk_ws(Wg1, as1, ad1), mk_ws(Wg2, as2, ad2)]

    u0 = u1 = None
    for li, ws in enumerate(layers):
        if li == 0:
            hwlo, hwhi, s1p, adp = _layer0(hpad0, *ws)
        else:
            hwlo, hwhi, s1p, adp = _layern(u0, u1, bts[li - 1], qpad, *ws)
        ex, den = _pass1(src3, dst3, s1p, adp)
        rec = _recip(den[0], den[1])
        up = _pass2(src3b, dst3b, hwlo, hwhi, ex, rec)
        u0, u1 = up[0], up[1]

    wf1 = jnp.zeros((HP, 384), f32).at[:FDIM, :2 * FDIM].set(Wf1)
    bf1p = jnp.zeros((8, 384), f32).at[0, :2 * FDIM].set(bf1)
    wf2 = jnp.zeros((384, NHID), f32).at[:2 * FDIM, :2 * NLABEL].set(Wf2)
    bf2p = jnp.zeros((8, NHID), f32).at[0, :2 * NLABEL].set(bf2)

    out = _final(u0, u1, bts[2], qpad, wf1, bf1p, wf2, bf2p)
    return out[:, :2 * NLABEL]
